# baseline (speedup 1.0000x reference)
import functools

import jax
import jax.numpy as jnp
from jax import lax
from jax.experimental import pallas as pl
from jax.experimental.pallas import tpu as pltpu
from jax.experimental.pallas import tpu_sc as plsc

_N = 50000
_NP = 50048
_D = 32
_NC = 2
_NS = 16
_NW = _NC * _NS
_L = 16
_B = 128
_IC = 512
_NOUT_U = 25088
_EPS = 1e-16

_SC_PARAMS = dict(needs_layout_passes=False, use_tc_tiling_on_sc=False)


def _conv_body(ept, nout, ndir, need_ex, epb, *refs):
    if need_ex:
        (x_ref, src_ref, dst_ref, numer_ref, denom_ref, ex_ref,
         acc_n, accd, sidx, didx, xs0, xs1, xd0, xd1, m, exb, zvec, pbuf,
         sem_s0, sem_s1, sem_d0, sem_d1) = refs
    else:
        (x_ref, src_ref, dst_ref, numer_ref, denom_ref,
         acc_n, accd, sidx, didx, xs0, xs1, xd0, xd1, m, exb, zvec, pbuf,
         sem_s0, sem_s1, sem_d0, sem_d1) = refs
        ex_ref = None
    xs = (xs0, xs1)
    xd = (xd0, xd1)
    sem_s = (sem_s0, sem_s1)
    sem_d = (sem_d0, sem_d1)

    c = lax.axis_index("c")
    s = lax.axis_index("s")
    tid = c * _NS + s

    zero16 = jnp.zeros((_L,), jnp.float32)

    def _zz(i, _):
        zvec[pl.ds(i * _L, _L)] = zero16
        return 0
    lax.fori_loop(0, 512 // _L, _zz, 0)

    def _zm(i, _):
        xs0[i, pl.ds(0, _L)] = zero16
        xs0[i, pl.ds(_L, _L)] = zero16
        return 0
    lax.fori_loop(0, _B, _zm, 0)

    r0 = s * (_N // _NS)
    def _zs(i, _):
        pltpu.sync_copy(xs0.at[pl.ds(0, 125)],
                        acc_n.at[pl.ds(r0 + i * 125, 125)])
        return 0
    lax.fori_loop(0, (_N // _NS) // 125, _zs, 0)
    d0 = s * (_NP // _NS)
    def _zd(i, _):
        pltpu.sync_copy(zvec, accd.at[pl.ds(d0 + i * 512, 512)])
        return 0
    lax.fori_loop(0, 6, _zd, 0)
    pltpu.sync_copy(zvec.at[pl.ds(0, 56)],
                    accd.at[pl.ds(d0 + 3072, 56)])
    plsc.subcore_barrier()

    iota16 = lax.iota(jnp.int32, _L)
    lane15 = iota16 * _L + (_L - 1)
    rows_per_chunk = _IC // _B

    for p in range(ndir):
        s_src = src_ref if p == 0 else dst_ref
        s_dst = dst_ref if p == 0 else src_ref

        def _chunk(ib, _, _p=p, _ss=s_src, _sd=s_dst):
            base_row = tid * (ept // _B) + ib * rows_per_chunk
            pltpu.sync_copy(_ss.at[pl.ds(base_row, rows_per_chunk)], sidx)
            pltpu.sync_copy(_sd.at[pl.ds(base_row, rows_per_chunk)], didx)
            pend = {}
            pend[0] = (
                pltpu.async_copy(x_ref.at[sidx.at[0]], xs[0], sem_s[0]),
                pltpu.async_copy(x_ref.at[didx.at[0]], xd[0], sem_d[0]),
            )
            for jb in range(rows_per_chunk):
                sl = jb % 2
                if jb + 1 < rows_per_chunk:
                    nsl = (jb + 1) % 2
                    pend[jb + 1] = (
                        pltpu.async_copy(x_ref.at[sidx.at[jb + 1]],
                                         xs[nsl], sem_s[nsl]),
                        pltpu.async_copy(x_ref.at[didx.at[jb + 1]],
                                         xd[nsl], sem_d[nsl]),
                    )
                pend[jb][0].wait()
                pend[jb][1].wait()
                _xs = xs[sl]
                _xd = xd[sl]

                def _group(g, _, _jb=jb, _xs=_xs, _xd=_xd):
                    off = g * _L
                    vs = sidx[_jb, pl.ds(off, _L)]
                    vd = didx[_jb, pl.ds(off, _L)]
                    for i in range(_L):
                        e = off + i
                        p16 = (_xs[e, pl.ds(0, _L)] * _xd[e, pl.ds(0, _L)] +
                               _xs[e, pl.ds(_L, _L)] * _xd[e, pl.ds(_L, _L)])
                        pbuf[pl.ds(i * _L, _L)] = plsc.cumsum(p16)
                    vdot = plsc.load_gather(pbuf, [lane15])
                    ex = jnp.where(vs != vd, jnp.exp(vdot), 0.0)
                    exb[_jb, pl.ds(off, _L)] = ex
                    for i in range(_L):
                        e = off + i
                        exi = ex[i]
                        m[e, pl.ds(0, _L)] = exi * _xs[e, pl.ds(0, _L)]
                        m[e, pl.ds(_L, _L)] = exi * _xs[e, pl.ds(_L, _L)]
                    return 0

                lax.fori_loop(0, _B // _L, _group, 0)
                pltpu.sync_copy(m, acc_n.at[didx.at[jb]], add=True)
                pltpu.sync_copy(exb.at[jb], accd.at[didx.at[jb]], add=True)
            if need_ex:
                pltpu.sync_copy(
                    exb, ex_ref.at[pl.ds(_p * epb + base_row, rows_per_chunk)])
            return 0

        lax.fori_loop(0, ept // _IC, _chunk, 0)

    plsc.subcore_barrier()

    dstripe = _NP // _NS
    pltpu.sync_copy(accd.at[pl.ds(s * dstripe, dstripe)],
                    denom_ref.at[c, pl.ds(s * dstripe, dstripe)])

    rows_per_tile = nout // _NS
    chunk = 112 if rows_per_tile % 125 else 125
    q0 = s * rows_per_tile
    def _cp(i, _):
        pltpu.sync_copy(acc_n.at[pl.ds(q0 + i * chunk, chunk)],
                        m.at[pl.ds(0, chunk)])
        pltpu.sync_copy(m.at[pl.ds(0, chunk)],
                        numer_ref.at[c, pl.ds(q0 + i * chunk, chunk)])
        return 0
    lax.fori_loop(0, rows_per_tile // chunk, _cp, 0)


def _make_conv(ept, nout, ndir, need_ex, epb):
    mesh = plsc.VectorSubcoreMesh(core_axis_name="c", subcore_axis_name="s",
                                  num_cores=_NC, num_subcores=_NS)
    outs = [
        jax.ShapeDtypeStruct((_NC, nout, _D), jnp.float32),
        jax.ShapeDtypeStruct((_NC, _NP), jnp.float32),
    ]
    if need_ex:
        outs.append(jax.ShapeDtypeStruct((ndir * epb, _B), jnp.float32))
    scratch = [
        pltpu.VMEM_SHARED((_N, _D), jnp.float32),
        pltpu.VMEM_SHARED((_NP,), jnp.float32),
        pltpu.VMEM((_IC // _B, _B), jnp.int32),
        pltpu.VMEM((_IC // _B, _B), jnp.int32),
        pltpu.VMEM((_B, _D), jnp.float32),
        pltpu.VMEM((_B, _D), jnp.float32),
        pltpu.VMEM((_B, _D), jnp.float32),
        pltpu.VMEM((_B, _D), jnp.float32),
        pltpu.VMEM((_B, _D), jnp.float32),
        pltpu.VMEM((_IC // _B, _B), jnp.float32),
        pltpu.VMEM((512,), jnp.float32),
        pltpu.VMEM((_B * 2,), jnp.float32),
        pltpu.SemaphoreType.DMA,
        pltpu.SemaphoreType.DMA,
        pltpu.SemaphoreType.DMA,
        pltpu.SemaphoreType.DMA,
    ]
    body = functools.partial(_conv_body, ept, nout, ndir, need_ex, epb)
    return pl.kernel(body, out_type=tuple(outs), mesh=mesh,
                     compiler_params=pltpu.CompilerParams(**_SC_PARAMS),
                     scratch_types=scratch)


def _alpha_body(ept, epb, den_ref, src_ref, dst_ref, ex_ref, al_ref,
                denl, didx, exb, alb):
    c = lax.axis_index("c")
    s = lax.axis_index("s")
    tid = c * _NS + s
    pltpu.sync_copy(den_ref, denl)
    rows_per_chunk = _IC // _B
    for p in range(2):
        dref = dst_ref if p == 0 else src_ref

        def _chunk(ib, _, _p=p, _dref=dref):
            base = tid * (ept // _B) + ib * rows_per_chunk
            pltpu.sync_copy(_dref.at[pl.ds(base, rows_per_chunk)], didx)
            pltpu.sync_copy(ex_ref.at[pl.ds(_p * epb + base, rows_per_chunk)],
                            exb)
            for jb in range(rows_per_chunk):
                def _grp(g, _, _jb=jb):
                    off = g * _L
                    vd = didx[_jb, pl.ds(off, _L)]
                    dv = plsc.load_gather(denl, [vd])
                    ev = exb[_jb, pl.ds(off, _L)]
                    alb[_jb, pl.ds(off, _L)] = ev / (dv + _EPS)
                    return 0
                lax.fori_loop(0, _B // _L, _grp, 0)
            pltpu.sync_copy(alb, al_ref.at[pl.ds(_p * epb + base,
                                                 rows_per_chunk)])
            return 0

        lax.fori_loop(0, ept // _IC, _chunk, 0)


def _make_alpha(ept, epb):
    mesh = plsc.VectorSubcoreMesh(core_axis_name="c", subcore_axis_name="s",
                                  num_cores=_NC, num_subcores=_NS)
    scratch = [
        pltpu.VMEM((_N,), jnp.float32),
        pltpu.VMEM((_IC // _B, _B), jnp.int32),
        pltpu.VMEM((_IC // _B, _B), jnp.float32),
        pltpu.VMEM((_IC // _B, _B), jnp.float32),
    ]
    body = functools.partial(_alpha_body, ept, epb)
    return pl.kernel(body,
                     out_type=jax.ShapeDtypeStruct((2 * epb, _B), jnp.float32),
                     mesh=mesh,
                     compiler_params=pltpu.CompilerParams(**_SC_PARAMS),
                     scratch_types=scratch)


_RB = 1000


def _prep_feats_body(f_ref, w_ref, b_ref, o_ref):
    y = lax.dot_general(f_ref[...], w_ref[...],
                        (((1,), (1,)), ((), ())),
                        preferred_element_type=jnp.float32)
    y = y + b_ref[...]
    y = jnp.where(y > 0, y, 0.01 * y)
    n = jnp.sqrt(jnp.sum(y * y, axis=1, keepdims=True))
    o_ref[...] = y / jnp.maximum(n, 1e-12)


def _norm_body(p_ref, o_ref):
    p = p_ref[...]
    n = jnp.sqrt(jnp.sum(p * p, axis=1, keepdims=True))
    o_ref[...] = p / jnp.maximum(n, 1e-12)


def _route_body(p_ref, n_ref, d_ref, o_ref):
    num = n_ref[0] + n_ref[1]
    den = d_ref[0, 0, 0, :] + d_ref[1, 0, 0, :]
    out = num / (den[:, None] + _EPS)
    p2 = p_ref[...] + out
    nn = jnp.sqrt(jnp.sum(p2 * p2, axis=1, keepdims=True))
    o_ref[...] = p2 / jnp.maximum(nn, 1e-12)


def _final_body(x_ref, n_ref, d_ref, y_ref, dt_ref):
    num = n_ref[0] + n_ref[1]
    den = d_ref[0, 0, 0, :] + d_ref[1, 0, 0, :]
    out = num / (den[:, None] + _EPS)
    out = jnp.where(out > 0, out, 0.01 * out)
    y_ref[...] = x_ref[...] + out
    dt_ref[...] = jnp.broadcast_to(den[None, None, None, :],
                                   (8, 1, 1, den.shape[0]))


def kernel(feature, edge_index, preference, W, b):
    nu = preference.shape[0]
    ni = feature.shape[0]
    n_nodes = nu + ni
    e = edge_index.shape[1]

    ep = -(-e // (_NW * _IC)) * (_NW * _IC)
    epb = ep // _B
    ept = ep // _NW
    pad = ep - e
    src = jnp.concatenate([edge_index[0], jnp.zeros((pad,), jnp.int32)])
    dst = jnp.concatenate([edge_index[1], jnp.zeros((pad,), jnp.int32)])
    src2d = src.reshape(epb, _B)
    dst2d = dst.reshape(epb, _B)

    feats = pl.pallas_call(
        _prep_feats_body,
        grid=(ni // _RB,),
        in_specs=[
            pl.BlockSpec((_RB, feature.shape[1]), lambda i: (i, 0)),
            pl.BlockSpec(W.shape, lambda i: (0, 0)),
            pl.BlockSpec((1, _D), lambda i: (0, 0)),
        ],
        out_specs=pl.BlockSpec((_RB, _D), lambda i: (i, 0)),
        out_shape=jax.ShapeDtypeStruct((ni, _D), jnp.float32),
    )(feature, W, b.reshape(1, _D))

    pref = pl.pallas_call(
        _norm_body,
        grid=(nu // _RB,),
        in_specs=[pl.BlockSpec((_RB, _D), lambda i: (i, 0))],
        out_specs=pl.BlockSpec((_RB, _D), lambda i: (i, 0)),
        out_shape=jax.ShapeDtypeStruct((nu, _D), jnp.float32),
    )(preference)

    conv_route = _make_conv(ept, _NOUT_U, 1, False, epb)
    route = pl.pallas_call(
        _route_body,
        grid=(nu // _RB,),
        in_specs=[
            pl.BlockSpec((_RB, _D), lambda i: (i, 0)),
            pl.BlockSpec((_NC, _RB, _D), lambda i: (0, i, 0)),
            pl.BlockSpec((_NC, 1, 1, _RB), lambda i: (0, i, 0, 0)),
        ],
        out_specs=pl.BlockSpec((_RB, _D), lambda i: (i, 0)),
        out_shape=jax.ShapeDtypeStruct((nu, _D), jnp.float32),
    )

    for _ in range(2):
        x = jnp.concatenate([pref, feats], axis=0)
        numer_p, denom_p = conv_route(x, src2d, dst2d)
        d3 = denom_p[:, :nu].reshape(_NC, nu // _RB, 1, _RB)
        pref = route(pref, numer_p[:, :nu], d3)

    x = jnp.concatenate([pref, feats], axis=0)
    conv_final = _make_conv(ept, n_nodes, 2, True, epb)
    numer_p, denom_p, ex2d = conv_final(x, src2d, dst2d)

    y, dt = pl.pallas_call(
        _final_body,
        grid=(n_nodes // _RB,),
        in_specs=[
            pl.BlockSpec((_RB, _D), lambda i: (i, 0)),
            pl.BlockSpec((_NC, _RB, _D), lambda i: (0, i, 0)),
            pl.BlockSpec((_NC, 1, 1, _RB), lambda i: (0, i, 0, 0)),
        ],
        out_specs=[
            pl.BlockSpec((_RB, _D), lambda i: (i, 0)),
            pl.BlockSpec((8, 1, 1, _RB), lambda i: (0, i, 0, 0)),
        ],
        out_shape=[
            jax.ShapeDtypeStruct((n_nodes, _D), jnp.float32),
            jax.ShapeDtypeStruct((8, n_nodes // _RB, 1, _RB), jnp.float32),
        ],
    )(x, numer_p, denom_p[:, :n_nodes].reshape(_NC, n_nodes // _RB, 1, _RB))

    al2d = _make_alpha(ept, epb)(dt[0].reshape(n_nodes), src2d, dst2d, ex2d)
    al = al2d.reshape(-1)
    alpha = jnp.concatenate([al[:e], al[ep:ep + e]])[:, None]
    return (y, alpha)


# speedup vs baseline: 1.0489x; 1.0489x over previous
import functools

import jax
import jax.numpy as jnp
from jax import lax
from jax.experimental import pallas as pl
from jax.experimental.pallas import tpu as pltpu
from jax.experimental.pallas import tpu_sc as plsc

_N = 50000
_NP = 50048
_D = 32
_NC = 2
_NS = 16
_NW = _NC * _NS
_L = 16
_B = 128
_IC = 512
_NOUT_U = 25088
_EPS = 1e-16

_SC_PARAMS = dict(needs_layout_passes=False, use_tc_tiling_on_sc=False)


def _conv_body(ept, nout, ndir, need_ex, epb, *refs):
    if need_ex:
        (x_ref, src_ref, dst_ref, numer_ref, denom_ref, ex_ref,
         acc_n, accd, sidx, didx, xs0, xs1, xd0, xd1, m0, m1, exb, zvec, pbuf,
         sem_s0, sem_s1, sem_d0, sem_d1, sem_m0, sem_m1, sem_e) = refs
    else:
        (x_ref, src_ref, dst_ref, numer_ref, denom_ref,
         acc_n, accd, sidx, didx, xs0, xs1, xd0, xd1, m0, m1, exb, zvec, pbuf,
         sem_s0, sem_s1, sem_d0, sem_d1, sem_m0, sem_m1, sem_e) = refs
        ex_ref = None
    xs = (xs0, xs1)
    xd = (xd0, xd1)
    mm = (m0, m1)
    sem_s = (sem_s0, sem_s1)
    sem_d = (sem_d0, sem_d1)
    sem_m = (sem_m0, sem_m1)

    c = lax.axis_index("c")
    s = lax.axis_index("s")
    tid = c * _NS + s

    zero16 = jnp.zeros((_L,), jnp.float32)

    def _zz(i, _):
        zvec[pl.ds(i * _L, _L)] = zero16
        return 0
    lax.fori_loop(0, 512 // _L, _zz, 0)

    def _zm(i, _):
        xs0[i, pl.ds(0, _L)] = zero16
        xs0[i, pl.ds(_L, _L)] = zero16
        return 0
    lax.fori_loop(0, _B, _zm, 0)

    r0 = s * (_N // _NS)
    def _zs(i, _):
        pltpu.sync_copy(xs0.at[pl.ds(0, 125)],
                        acc_n.at[pl.ds(r0 + i * 125, 125)])
        return 0
    lax.fori_loop(0, (_N // _NS) // 125, _zs, 0)
    d0 = s * (_NP // _NS)
    def _zd(i, _):
        pltpu.sync_copy(zvec, accd.at[pl.ds(d0 + i * 512, 512)])
        return 0
    lax.fori_loop(0, 6, _zd, 0)
    pltpu.sync_copy(zvec.at[pl.ds(0, 56)],
                    accd.at[pl.ds(d0 + 3072, 56)])
    plsc.subcore_barrier()

    iota16 = lax.iota(jnp.int32, _L)
    lane15 = iota16 * _L + (_L - 1)
    rows_per_chunk = _IC // _B

    for p in range(ndir):
        s_src = src_ref if p == 0 else dst_ref
        s_dst = dst_ref if p == 0 else src_ref

        def _chunk(ib, _, _p=p, _ss=s_src, _sd=s_dst):
            base_row = tid * (ept // _B) + ib * rows_per_chunk
            pltpu.sync_copy(_ss.at[pl.ds(base_row, rows_per_chunk)], sidx)
            pltpu.sync_copy(_sd.at[pl.ds(base_row, rows_per_chunk)], didx)
            pend = {}
            pend[0] = (
                pltpu.async_copy(x_ref.at[sidx.at[0]], xs[0], sem_s[0]),
                pltpu.async_copy(x_ref.at[didx.at[0]], xd[0], sem_d[0]),
            )
            scat = {}
            for jb in range(rows_per_chunk):
                sl = jb % 2
                if jb + 1 < rows_per_chunk:
                    nsl = (jb + 1) % 2
                    pend[jb + 1] = (
                        pltpu.async_copy(x_ref.at[sidx.at[jb + 1]],
                                         xs[nsl], sem_s[nsl]),
                        pltpu.async_copy(x_ref.at[didx.at[jb + 1]],
                                         xd[nsl], sem_d[nsl]),
                    )
                pend[jb][0].wait()
                pend[jb][1].wait()
                if jb >= 2:
                    scat[jb - 2].wait()
                _xs = xs[sl]
                _xd = xd[sl]
                m = mm[sl]

                def _group(g, _, _jb=jb, _xs=_xs, _xd=_xd, m=m):
                    off = g * _L
                    vs = sidx[_jb, pl.ds(off, _L)]
                    vd = didx[_jb, pl.ds(off, _L)]
                    for i in range(_L):
                        e = off + i
                        p16 = (_xs[e, pl.ds(0, _L)] * _xd[e, pl.ds(0, _L)] +
                               _xs[e, pl.ds(_L, _L)] * _xd[e, pl.ds(_L, _L)])
                        pbuf[pl.ds(i * _L, _L)] = plsc.cumsum(p16)
                    vdot = plsc.load_gather(pbuf, [lane15])
                    ex = jnp.where(vs != vd, jnp.exp(vdot), 0.0)
                    exb[_jb, pl.ds(off, _L)] = ex
                    for i in range(_L):
                        e = off + i
                        exi = ex[i]
                        m[e, pl.ds(0, _L)] = exi * _xs[e, pl.ds(0, _L)]
                        m[e, pl.ds(_L, _L)] = exi * _xs[e, pl.ds(_L, _L)]
                    return 0

                lax.fori_loop(0, _B // _L, _group, 0)
                scat[jb] = pltpu.async_copy(m, acc_n.at[didx.at[jb]],
                                            sem_m[sl], add=True)
                scat[(jb, "e")] = pltpu.async_copy(
                    exb.at[jb], accd.at[didx.at[jb]], sem_e, add=True)
            scat[rows_per_chunk - 2].wait()
            scat[rows_per_chunk - 1].wait()
            for jb in range(rows_per_chunk):
                scat[(jb, "e")].wait()
            if need_ex:
                pltpu.sync_copy(
                    exb, ex_ref.at[pl.ds(_p * epb + base_row, rows_per_chunk)])
            return 0

        lax.fori_loop(0, ept // _IC, _chunk, 0)

    plsc.subcore_barrier()

    dstripe = _NP // _NS
    pltpu.sync_copy(accd.at[pl.ds(s * dstripe, dstripe)],
                    denom_ref.at[c, pl.ds(s * dstripe, dstripe)])

    rows_per_tile = nout // _NS
    chunk = 112 if rows_per_tile % 125 else 125
    q0 = s * rows_per_tile
    def _cp(i, _):
        pltpu.sync_copy(acc_n.at[pl.ds(q0 + i * chunk, chunk)],
                        m0.at[pl.ds(0, chunk)])
        pltpu.sync_copy(m0.at[pl.ds(0, chunk)],
                        numer_ref.at[c, pl.ds(q0 + i * chunk, chunk)])
        return 0
    lax.fori_loop(0, rows_per_tile // chunk, _cp, 0)


def _make_conv(ept, nout, ndir, need_ex, epb):
    mesh = plsc.VectorSubcoreMesh(core_axis_name="c", subcore_axis_name="s",
                                  num_cores=_NC, num_subcores=_NS)
    outs = [
        jax.ShapeDtypeStruct((_NC, nout, _D), jnp.float32),
        jax.ShapeDtypeStruct((_NC, _NP), jnp.float32),
    ]
    if need_ex:
        outs.append(jax.ShapeDtypeStruct((ndir * epb, _B), jnp.float32))
    scratch = [
        pltpu.VMEM_SHARED((_N, _D), jnp.float32),
        pltpu.VMEM_SHARED((_NP,), jnp.float32),
        pltpu.VMEM((_IC // _B, _B), jnp.int32),
        pltpu.VMEM((_IC // _B, _B), jnp.int32),
        pltpu.VMEM((_B, _D), jnp.float32),
        pltpu.VMEM((_B, _D), jnp.float32),
        pltpu.VMEM((_B, _D), jnp.float32),
        pltpu.VMEM((_B, _D), jnp.float32),
        pltpu.VMEM((_B, _D), jnp.float32),
        pltpu.VMEM((_B, _D), jnp.float32),
        pltpu.VMEM((_IC // _B, _B), jnp.float32),
        pltpu.VMEM((512,), jnp.float32),
        pltpu.VMEM((_B * 2,), jnp.float32),
        pltpu.SemaphoreType.DMA,
        pltpu.SemaphoreType.DMA,
        pltpu.SemaphoreType.DMA,
        pltpu.SemaphoreType.DMA,
        pltpu.SemaphoreType.DMA,
        pltpu.SemaphoreType.DMA,
        pltpu.SemaphoreType.DMA,
    ]
    body = functools.partial(_conv_body, ept, nout, ndir, need_ex, epb)
    return pl.kernel(body, out_type=tuple(outs), mesh=mesh,
                     compiler_params=pltpu.CompilerParams(**_SC_PARAMS),
                     scratch_types=scratch)


def _alpha_body(ept, epb, den_ref, src_ref, dst_ref, ex_ref, al_ref,
                denl, didx, exb, alb):
    c = lax.axis_index("c")
    s = lax.axis_index("s")
    tid = c * _NS + s
    pltpu.sync_copy(den_ref, denl)
    rows_per_chunk = _IC // _B
    for p in range(2):
        dref = dst_ref if p == 0 else src_ref

        def _chunk(ib, _, _p=p, _dref=dref):
            base = tid * (ept // _B) + ib * rows_per_chunk
            pltpu.sync_copy(_dref.at[pl.ds(base, rows_per_chunk)], didx)
            pltpu.sync_copy(ex_ref.at[pl.ds(_p * epb + base, rows_per_chunk)],
                            exb)
            for jb in range(rows_per_chunk):
                def _grp(g, _, _jb=jb):
                    off = g * _L
                    vd = didx[_jb, pl.ds(off, _L)]
                    dv = plsc.load_gather(denl, [vd])
                    ev = exb[_jb, pl.ds(off, _L)]
                    alb[_jb, pl.ds(off, _L)] = ev / (dv + _EPS)
                    return 0
                lax.fori_loop(0, _B // _L, _grp, 0)
            pltpu.sync_copy(alb, al_ref.at[pl.ds(_p * epb + base,
                                                 rows_per_chunk)])
            return 0

        lax.fori_loop(0, ept // _IC, _chunk, 0)


def _make_alpha(ept, epb):
    mesh = plsc.VectorSubcoreMesh(core_axis_name="c", subcore_axis_name="s",
                                  num_cores=_NC, num_subcores=_NS)
    scratch = [
        pltpu.VMEM((_N,), jnp.float32),
        pltpu.VMEM((_IC // _B, _B), jnp.int32),
        pltpu.VMEM((_IC // _B, _B), jnp.float32),
        pltpu.VMEM((_IC // _B, _B), jnp.float32),
    ]
    body = functools.partial(_alpha_body, ept, epb)
    return pl.kernel(body,
                     out_type=jax.ShapeDtypeStruct((2 * epb, _B), jnp.float32),
                     mesh=mesh,
                     compiler_params=pltpu.CompilerParams(**_SC_PARAMS),
                     scratch_types=scratch)


_RB = 1000


def _prep_feats_body(f_ref, w_ref, b_ref, o_ref):
    y = lax.dot_general(f_ref[...], w_ref[...],
                        (((1,), (1,)), ((), ())),
                        preferred_element_type=jnp.float32)
    y = y + b_ref[...]
    y = jnp.where(y > 0, y, 0.01 * y)
    n = jnp.sqrt(jnp.sum(y * y, axis=1, keepdims=True))
    o_ref[...] = y / jnp.maximum(n, 1e-12)


def _norm_body(p_ref, o_ref):
    p = p_ref[...]
    n = jnp.sqrt(jnp.sum(p * p, axis=1, keepdims=True))
    o_ref[...] = p / jnp.maximum(n, 1e-12)


def _route_body(p_ref, n_ref, d_ref, o_ref):
    num = n_ref[0] + n_ref[1]
    den = d_ref[0, 0, 0, :] + d_ref[1, 0, 0, :]
    out = num / (den[:, None] + _EPS)
    p2 = p_ref[...] + out
    nn = jnp.sqrt(jnp.sum(p2 * p2, axis=1, keepdims=True))
    o_ref[...] = p2 / jnp.maximum(nn, 1e-12)


def _final_body(x_ref, n_ref, d_ref, y_ref, dt_ref):
    num = n_ref[0] + n_ref[1]
    den = d_ref[0, 0, 0, :] + d_ref[1, 0, 0, :]
    out = num / (den[:, None] + _EPS)
    out = jnp.where(out > 0, out, 0.01 * out)
    y_ref[...] = x_ref[...] + out
    dt_ref[...] = jnp.broadcast_to(den[None, None, None, :],
                                   (8, 1, 1, den.shape[0]))


def kernel(feature, edge_index, preference, W, b):
    nu = preference.shape[0]
    ni = feature.shape[0]
    n_nodes = nu + ni
    e = edge_index.shape[1]

    ep = -(-e // (_NW * _IC)) * (_NW * _IC)
    epb = ep // _B
    ept = ep // _NW
    pad = ep - e
    src = jnp.concatenate([edge_index[0], jnp.zeros((pad,), jnp.int32)])
    dst = jnp.concatenate([edge_index[1], jnp.zeros((pad,), jnp.int32)])
    src2d = src.reshape(epb, _B)
    dst2d = dst.reshape(epb, _B)

    feats = pl.pallas_call(
        _prep_feats_body,
        grid=(ni // _RB,),
        in_specs=[
            pl.BlockSpec((_RB, feature.shape[1]), lambda i: (i, 0)),
            pl.BlockSpec(W.shape, lambda i: (0, 0)),
            pl.BlockSpec((1, _D), lambda i: (0, 0)),
        ],
        out_specs=pl.BlockSpec((_RB, _D), lambda i: (i, 0)),
        out_shape=jax.ShapeDtypeStruct((ni, _D), jnp.float32),
    )(feature, W, b.reshape(1, _D))

    pref = pl.pallas_call(
        _norm_body,
        grid=(nu // _RB,),
        in_specs=[pl.BlockSpec((_RB, _D), lambda i: (i, 0))],
        out_specs=pl.BlockSpec((_RB, _D), lambda i: (i, 0)),
        out_shape=jax.ShapeDtypeStruct((nu, _D), jnp.float32),
    )(preference)

    conv_route = _make_conv(ept, _NOUT_U, 1, False, epb)
    route = pl.pallas_call(
        _route_body,
        grid=(nu // _RB,),
        in_specs=[
            pl.BlockSpec((_RB, _D), lambda i: (i, 0)),
            pl.BlockSpec((_NC, _RB, _D), lambda i: (0, i, 0)),
            pl.BlockSpec((_NC, 1, 1, _RB), lambda i: (0, i, 0, 0)),
        ],
        out_specs=pl.BlockSpec((_RB, _D), lambda i: (i, 0)),
        out_shape=jax.ShapeDtypeStruct((nu, _D), jnp.float32),
    )

    for _ in range(2):
        x = jnp.concatenate([pref, feats], axis=0)
        numer_p, denom_p = conv_route(x, src2d, dst2d)
        d3 = denom_p[:, :nu].reshape(_NC, nu // _RB, 1, _RB)
        pref = route(pref, numer_p[:, :nu], d3)

    x = jnp.concatenate([pref, feats], axis=0)
    conv_final = _make_conv(ept, n_nodes, 2, True, epb)
    numer_p, denom_p, ex2d = conv_final(x, src2d, dst2d)

    y, dt = pl.pallas_call(
        _final_body,
        grid=(n_nodes // _RB,),
        in_specs=[
            pl.BlockSpec((_RB, _D), lambda i: (i, 0)),
            pl.BlockSpec((_NC, _RB, _D), lambda i: (0, i, 0)),
            pl.BlockSpec((_NC, 1, 1, _RB), lambda i: (0, i, 0, 0)),
        ],
        out_specs=[
            pl.BlockSpec((_RB, _D), lambda i: (i, 0)),
            pl.BlockSpec((8, 1, 1, _RB), lambda i: (0, i, 0, 0)),
        ],
        out_shape=[
            jax.ShapeDtypeStruct((n_nodes, _D), jnp.float32),
            jax.ShapeDtypeStruct((8, n_nodes // _RB, 1, _RB), jnp.float32),
        ],
    )(x, numer_p, denom_p[:, :n_nodes].reshape(_NC, n_nodes // _RB, 1, _RB))

    al2d = _make_alpha(ept, epb)(dt[0].reshape(n_nodes), src2d, dst2d, ex2d)
    al = al2d.reshape(-1)
    alpha = jnp.concatenate([al[:e], al[ep:ep + e]])[:, None]
    return (y, alpha)


# speedup vs baseline: 1.3579x; 1.2946x over previous
import functools

import jax
import jax.numpy as jnp
from jax import lax
from jax.experimental import pallas as pl
from jax.experimental.pallas import tpu as pltpu
from jax.experimental.pallas import tpu_sc as plsc

_N = 50000
_NP = 50048
_D = 32
_NC = 2
_NS = 16
_NW = _NC * _NS
_L = 16
_B = 128
_IC = 512
_NOUT_U = 25088
_EPS = 1e-16

_SC_PARAMS = dict(needs_layout_passes=False, use_tc_tiling_on_sc=False)


def _conv_body(ept, nout, ndir, need_ex, epb, *refs):
    if need_ex:
        (x_ref, src_ref, dst_ref, numer_ref, denom_ref, ex_ref,
         acc_n, accd, sidx, didx, xs0, xs1, xd0, xd1, m0, m1, exb, zvec, pbuf,
         sem_s0, sem_s1, sem_d0, sem_d1, sem_m0, sem_m1, sem_e) = refs
    else:
        (x_ref, src_ref, dst_ref, numer_ref, denom_ref,
         acc_n, accd, sidx, didx, xs0, xs1, xd0, xd1, m0, m1, exb, zvec, pbuf,
         sem_s0, sem_s1, sem_d0, sem_d1, sem_m0, sem_m1, sem_e) = refs
        ex_ref = None
    xs = (xs0, xs1)
    xd = (xd0, xd1)
    mm = (m0, m1)
    sem_s = (sem_s0, sem_s1)
    sem_d = (sem_d0, sem_d1)
    sem_m = (sem_m0, sem_m1)

    c = lax.axis_index("c")
    s = lax.axis_index("s")
    tid = c * _NS + s

    zero16 = jnp.zeros((_L,), jnp.float32)

    def _zz(i, _):
        zvec[pl.ds(i * _L, _L)] = zero16
        return 0
    lax.fori_loop(0, 512 // _L, _zz, 0)

    def _zm(i, _):
        xs0[i, pl.ds(0, _L)] = zero16
        xs0[i, pl.ds(_L, _L)] = zero16
        return 0
    lax.fori_loop(0, _B, _zm, 0)

    r0 = s * (_N // _NS)
    def _zs(i, _):
        pltpu.sync_copy(xs0.at[pl.ds(0, 125)],
                        acc_n.at[pl.ds(r0 + i * 125, 125)])
        return 0
    lax.fori_loop(0, (_N // _NS) // 125, _zs, 0)
    d0 = s * (_NP // _NS)
    def _zd(i, _):
        pltpu.sync_copy(zvec, accd.at[pl.ds(d0 + i * 512, 512)])
        return 0
    lax.fori_loop(0, 6, _zd, 0)
    pltpu.sync_copy(zvec.at[pl.ds(0, 56)],
                    accd.at[pl.ds(d0 + 3072, 56)])
    plsc.subcore_barrier()

    iota16 = lax.iota(jnp.int32, _L)
    lane15 = iota16 * _L + (_L - 1)
    rows_per_chunk = _IC // _B

    if ndir == 2:
        def _chunk2(ib, _):
            base_row = tid * (ept // _B) + ib * rows_per_chunk
            pltpu.sync_copy(src_ref.at[pl.ds(base_row, rows_per_chunk)], sidx)
            pltpu.sync_copy(dst_ref.at[pl.ds(base_row, rows_per_chunk)], didx)
            pend = {}
            pend[0] = (
                pltpu.async_copy(x_ref.at[sidx.at[0]], xs[0], sem_s[0]),
                pltpu.async_copy(x_ref.at[didx.at[0]], xd[0], sem_d[0]),
            )
            scat = {}
            for jb in range(rows_per_chunk):
                sl = jb % 2
                if jb + 1 < rows_per_chunk:
                    nsl = (jb + 1) % 2
                    pend[jb + 1] = (
                        pltpu.async_copy(x_ref.at[sidx.at[jb + 1]],
                                         xs[nsl], sem_s[nsl]),
                        pltpu.async_copy(x_ref.at[didx.at[jb + 1]],
                                         xd[nsl], sem_d[nsl]),
                    )
                pend[jb][0].wait()
                pend[jb][1].wait()
                if jb >= 1:
                    scat[(jb - 1, "f")].wait()
                    scat[(jb - 1, "b")].wait()

                def _group2(g, _, _jb=jb, _xs=xs[sl], _xd=xd[sl]):
                    off = g * _L
                    vs = sidx[_jb, pl.ds(off, _L)]
                    vd = didx[_jb, pl.ds(off, _L)]
                    for i in range(_L):
                        e = off + i
                        p16 = (_xs[e, pl.ds(0, _L)] * _xd[e, pl.ds(0, _L)] +
                               _xs[e, pl.ds(_L, _L)] * _xd[e, pl.ds(_L, _L)])
                        pbuf[pl.ds(i * _L, _L)] = plsc.cumsum(p16)
                    vdot = plsc.load_gather(pbuf, [lane15])
                    ex = jnp.where(vs != vd, jnp.exp(vdot), 0.0)
                    exb[_jb, pl.ds(off, _L)] = ex
                    for i in range(_L):
                        e = off + i
                        exi = ex[i]
                        m0[e, pl.ds(0, _L)] = exi * _xs[e, pl.ds(0, _L)]
                        m0[e, pl.ds(_L, _L)] = exi * _xs[e, pl.ds(_L, _L)]
                        m1[e, pl.ds(0, _L)] = exi * _xd[e, pl.ds(0, _L)]
                        m1[e, pl.ds(_L, _L)] = exi * _xd[e, pl.ds(_L, _L)]
                    return 0

                lax.fori_loop(0, _B // _L, _group2, 0)
                scat[(jb, "f")] = pltpu.async_copy(
                    m0, acc_n.at[didx.at[jb]], sem_m[0], add=True)
                scat[(jb, "b")] = pltpu.async_copy(
                    m1, acc_n.at[sidx.at[jb]], sem_m[1], add=True)
                scat[(jb, "e1")] = pltpu.async_copy(
                    exb.at[jb], accd.at[didx.at[jb]], sem_e, add=True)
                scat[(jb, "e2")] = pltpu.async_copy(
                    exb.at[jb], accd.at[sidx.at[jb]], sem_e, add=True)
            scat[(rows_per_chunk - 1, "f")].wait()
            scat[(rows_per_chunk - 1, "b")].wait()
            for jb in range(rows_per_chunk):
                scat[(jb, "e1")].wait()
                scat[(jb, "e2")].wait()
            pltpu.sync_copy(exb, ex_ref.at[pl.ds(base_row, rows_per_chunk)])
            pltpu.sync_copy(exb,
                            ex_ref.at[pl.ds(epb + base_row, rows_per_chunk)])
            return 0

        lax.fori_loop(0, ept // _IC, _chunk2, 0)

    for p in range(ndir if ndir == 1 else 0):
        s_src = src_ref if p == 0 else dst_ref
        s_dst = dst_ref if p == 0 else src_ref

        def _chunk(ib, _, _p=p, _ss=s_src, _sd=s_dst):
            base_row = tid * (ept // _B) + ib * rows_per_chunk
            pltpu.sync_copy(_ss.at[pl.ds(base_row, rows_per_chunk)], sidx)
            pltpu.sync_copy(_sd.at[pl.ds(base_row, rows_per_chunk)], didx)
            pend = {}
            pend[0] = (
                pltpu.async_copy(x_ref.at[sidx.at[0]], xs[0], sem_s[0]),
                pltpu.async_copy(x_ref.at[didx.at[0]], xd[0], sem_d[0]),
            )
            scat = {}
            for jb in range(rows_per_chunk):
                sl = jb % 2
                if jb + 1 < rows_per_chunk:
                    nsl = (jb + 1) % 2
                    pend[jb + 1] = (
                        pltpu.async_copy(x_ref.at[sidx.at[jb + 1]],
                                         xs[nsl], sem_s[nsl]),
                        pltpu.async_copy(x_ref.at[didx.at[jb + 1]],
                                         xd[nsl], sem_d[nsl]),
                    )
                pend[jb][0].wait()
                pend[jb][1].wait()
                if jb >= 2:
                    scat[jb - 2].wait()
                _xs = xs[sl]
                _xd = xd[sl]
                m = mm[sl]

                def _group(g, _, _jb=jb, _xs=_xs, _xd=_xd, m=m):
                    off = g * _L
                    vs = sidx[_jb, pl.ds(off, _L)]
                    vd = didx[_jb, pl.ds(off, _L)]
                    for i in range(_L):
                        e = off + i
                        p16 = (_xs[e, pl.ds(0, _L)] * _xd[e, pl.ds(0, _L)] +
                               _xs[e, pl.ds(_L, _L)] * _xd[e, pl.ds(_L, _L)])
                        pbuf[pl.ds(i * _L, _L)] = plsc.cumsum(p16)
                    vdot = plsc.load_gather(pbuf, [lane15])
                    ex = jnp.where(vs != vd, jnp.exp(vdot), 0.0)
                    exb[_jb, pl.ds(off, _L)] = ex
                    for i in range(_L):
                        e = off + i
                        exi = ex[i]
                        m[e, pl.ds(0, _L)] = exi * _xs[e, pl.ds(0, _L)]
                        m[e, pl.ds(_L, _L)] = exi * _xs[e, pl.ds(_L, _L)]
                    return 0

                lax.fori_loop(0, _B // _L, _group, 0)
                scat[jb] = pltpu.async_copy(m, acc_n.at[didx.at[jb]],
                                            sem_m[sl], add=True)
                scat[(jb, "e")] = pltpu.async_copy(
                    exb.at[jb], accd.at[didx.at[jb]], sem_e, add=True)
            scat[rows_per_chunk - 2].wait()
            scat[rows_per_chunk - 1].wait()
            for jb in range(rows_per_chunk):
                scat[(jb, "e")].wait()
            if need_ex:
                pltpu.sync_copy(
                    exb, ex_ref.at[pl.ds(_p * epb + base_row, rows_per_chunk)])
            return 0

        lax.fori_loop(0, ept // _IC, _chunk, 0)

    plsc.subcore_barrier()

    dstripe = _NP // _NS
    pltpu.sync_copy(accd.at[pl.ds(s * dstripe, dstripe)],
                    denom_ref.at[c, pl.ds(s * dstripe, dstripe)])

    rows_per_tile = nout // _NS
    chunk = 112 if rows_per_tile % 125 else 125
    q0 = s * rows_per_tile
    def _cp(i, _):
        pltpu.sync_copy(acc_n.at[pl.ds(q0 + i * chunk, chunk)],
                        m0.at[pl.ds(0, chunk)])
        pltpu.sync_copy(m0.at[pl.ds(0, chunk)],
                        numer_ref.at[c, pl.ds(q0 + i * chunk, chunk)])
        return 0
    lax.fori_loop(0, rows_per_tile // chunk, _cp, 0)


def _make_conv(ept, nout, ndir, need_ex, epb):
    mesh = plsc.VectorSubcoreMesh(core_axis_name="c", subcore_axis_name="s",
                                  num_cores=_NC, num_subcores=_NS)
    outs = [
        jax.ShapeDtypeStruct((_NC, nout, _D), jnp.float32),
        jax.ShapeDtypeStruct((_NC, _NP), jnp.float32),
    ]
    if need_ex:
        outs.append(jax.ShapeDtypeStruct((ndir * epb, _B), jnp.float32))
    scratch = [
        pltpu.VMEM_SHARED((_N, _D), jnp.float32),
        pltpu.VMEM_SHARED((_NP,), jnp.float32),
        pltpu.VMEM((_IC // _B, _B), jnp.int32),
        pltpu.VMEM((_IC // _B, _B), jnp.int32),
        pltpu.VMEM((_B, _D), jnp.float32),
        pltpu.VMEM((_B, _D), jnp.float32),
        pltpu.VMEM((_B, _D), jnp.float32),
        pltpu.VMEM((_B, _D), jnp.float32),
        pltpu.VMEM((_B, _D), jnp.float32),
        pltpu.VMEM((_B, _D), jnp.float32),
        pltpu.VMEM((_IC // _B, _B), jnp.float32),
        pltpu.VMEM((512,), jnp.float32),
        pltpu.VMEM((_B * 2,), jnp.float32),
        pltpu.SemaphoreType.DMA,
        pltpu.SemaphoreType.DMA,
        pltpu.SemaphoreType.DMA,
        pltpu.SemaphoreType.DMA,
        pltpu.SemaphoreType.DMA,
        pltpu.SemaphoreType.DMA,
        pltpu.SemaphoreType.DMA,
    ]
    body = functools.partial(_conv_body, ept, nout, ndir, need_ex, epb)
    return pl.kernel(body, out_type=tuple(outs), mesh=mesh,
                     compiler_params=pltpu.CompilerParams(**_SC_PARAMS),
                     scratch_types=scratch)


def _alpha_body(ept, epb, den_ref, src_ref, dst_ref, ex_ref, al_ref,
                denl, didx, exb, alb):
    c = lax.axis_index("c")
    s = lax.axis_index("s")
    tid = c * _NS + s
    pltpu.sync_copy(den_ref, denl)
    rows_per_chunk = _IC // _B
    for p in range(2):
        dref = dst_ref if p == 0 else src_ref

        def _chunk(ib, _, _p=p, _dref=dref):
            base = tid * (ept // _B) + ib * rows_per_chunk
            pltpu.sync_copy(_dref.at[pl.ds(base, rows_per_chunk)], didx)
            pltpu.sync_copy(ex_ref.at[pl.ds(_p * epb + base, rows_per_chunk)],
                            exb)
            for jb in range(rows_per_chunk):
                def _grp(g, _, _jb=jb):
                    off = g * _L
                    vd = didx[_jb, pl.ds(off, _L)]
                    dv = plsc.load_gather(denl, [vd])
                    ev = exb[_jb, pl.ds(off, _L)]
                    alb[_jb, pl.ds(off, _L)] = ev / (dv + _EPS)
                    return 0
                lax.fori_loop(0, _B // _L, _grp, 0)
            pltpu.sync_copy(alb, al_ref.at[pl.ds(_p * epb + base,
                                                 rows_per_chunk)])
            return 0

        lax.fori_loop(0, ept // _IC, _chunk, 0)


def _make_alpha(ept, epb):
    mesh = plsc.VectorSubcoreMesh(core_axis_name="c", subcore_axis_name="s",
                                  num_cores=_NC, num_subcores=_NS)
    scratch = [
        pltpu.VMEM((_N,), jnp.float32),
        pltpu.VMEM((_IC // _B, _B), jnp.int32),
        pltpu.VMEM((_IC // _B, _B), jnp.float32),
        pltpu.VMEM((_IC // _B, _B), jnp.float32),
    ]
    body = functools.partial(_alpha_body, ept, epb)
    return pl.kernel(body,
                     out_type=jax.ShapeDtypeStruct((2 * epb, _B), jnp.float32),
                     mesh=mesh,
                     compiler_params=pltpu.CompilerParams(**_SC_PARAMS),
                     scratch_types=scratch)


_RB = 1000


def _prep_feats_body(f_ref, w_ref, b_ref, o_ref):
    y = lax.dot_general(f_ref[...], w_ref[...],
                        (((1,), (1,)), ((), ())),
                        preferred_element_type=jnp.float32)
    y = y + b_ref[...]
    y = jnp.where(y > 0, y, 0.01 * y)
    n = jnp.sqrt(jnp.sum(y * y, axis=1, keepdims=True))
    o_ref[...] = y / jnp.maximum(n, 1e-12)


def _norm_body(p_ref, o_ref):
    p = p_ref[...]
    n = jnp.sqrt(jnp.sum(p * p, axis=1, keepdims=True))
    o_ref[...] = p / jnp.maximum(n, 1e-12)


def _route_body(p_ref, n_ref, d_ref, o_ref):
    num = n_ref[0] + n_ref[1]
    den = d_ref[0, 0, 0, :] + d_ref[1, 0, 0, :]
    out = num / (den[:, None] + _EPS)
    p2 = p_ref[...] + out
    nn = jnp.sqrt(jnp.sum(p2 * p2, axis=1, keepdims=True))
    o_ref[...] = p2 / jnp.maximum(nn, 1e-12)


def _final_body(x_ref, n_ref, d_ref, y_ref, dt_ref):
    num = n_ref[0] + n_ref[1]
    den = d_ref[0, 0, 0, :] + d_ref[1, 0, 0, :]
    out = num / (den[:, None] + _EPS)
    out = jnp.where(out > 0, out, 0.01 * out)
    y_ref[...] = x_ref[...] + out
    dt_ref[...] = jnp.broadcast_to(den[None, None, None, :],
                                   (8, 1, 1, den.shape[0]))


def kernel(feature, edge_index, preference, W, b):
    nu = preference.shape[0]
    ni = feature.shape[0]
    n_nodes = nu + ni
    e = edge_index.shape[1]

    ep = -(-e // (_NW * _IC)) * (_NW * _IC)
    epb = ep // _B
    ept = ep // _NW
    pad = ep - e
    src = jnp.concatenate([edge_index[0], jnp.zeros((pad,), jnp.int32)])
    dst = jnp.concatenate([edge_index[1], jnp.zeros((pad,), jnp.int32)])
    src2d = src.reshape(epb, _B)
    dst2d = dst.reshape(epb, _B)

    feats = pl.pallas_call(
        _prep_feats_body,
        grid=(ni // _RB,),
        in_specs=[
            pl.BlockSpec((_RB, feature.shape[1]), lambda i: (i, 0)),
            pl.BlockSpec(W.shape, lambda i: (0, 0)),
            pl.BlockSpec((1, _D), lambda i: (0, 0)),
        ],
        out_specs=pl.BlockSpec((_RB, _D), lambda i: (i, 0)),
        out_shape=jax.ShapeDtypeStruct((ni, _D), jnp.float32),
    )(feature, W, b.reshape(1, _D))

    pref = pl.pallas_call(
        _norm_body,
        grid=(nu // _RB,),
        in_specs=[pl.BlockSpec((_RB, _D), lambda i: (i, 0))],
        out_specs=pl.BlockSpec((_RB, _D), lambda i: (i, 0)),
        out_shape=jax.ShapeDtypeStruct((nu, _D), jnp.float32),
    )(preference)

    conv_route = _make_conv(ept, _NOUT_U, 1, False, epb)
    route = pl.pallas_call(
        _route_body,
        grid=(nu // _RB,),
        in_specs=[
            pl.BlockSpec((_RB, _D), lambda i: (i, 0)),
            pl.BlockSpec((_NC, _RB, _D), lambda i: (0, i, 0)),
            pl.BlockSpec((_NC, 1, 1, _RB), lambda i: (0, i, 0, 0)),
        ],
        out_specs=pl.BlockSpec((_RB, _D), lambda i: (i, 0)),
        out_shape=jax.ShapeDtypeStruct((nu, _D), jnp.float32),
    )

    for _ in range(2):
        x = jnp.concatenate([pref, feats], axis=0)
        numer_p, denom_p = conv_route(x, src2d, dst2d)
        d3 = denom_p[:, :nu].reshape(_NC, nu // _RB, 1, _RB)
        pref = route(pref, numer_p[:, :nu], d3)

    x = jnp.concatenate([pref, feats], axis=0)
    conv_final = _make_conv(ept, n_nodes, 2, True, epb)
    numer_p, denom_p, ex2d = conv_final(x, src2d, dst2d)

    y, dt = pl.pallas_call(
        _final_body,
        grid=(n_nodes // _RB,),
        in_specs=[
            pl.BlockSpec((_RB, _D), lambda i: (i, 0)),
            pl.BlockSpec((_NC, _RB, _D), lambda i: (0, i, 0)),
            pl.BlockSpec((_NC, 1, 1, _RB), lambda i: (0, i, 0, 0)),
        ],
        out_specs=[
            pl.BlockSpec((_RB, _D), lambda i: (i, 0)),
            pl.BlockSpec((8, 1, 1, _RB), lambda i: (0, i, 0, 0)),
        ],
        out_shape=[
            jax.ShapeDtypeStruct((n_nodes, _D), jnp.float32),
            jax.ShapeDtypeStruct((8, n_nodes // _RB, 1, _RB), jnp.float32),
        ],
    )(x, numer_p, denom_p[:, :n_nodes].reshape(_NC, n_nodes // _RB, 1, _RB))

    al2d = _make_alpha(ept, epb)(dt[0].reshape(n_nodes), src2d, dst2d, ex2d)
    al = al2d.reshape(-1)
    alpha = jnp.concatenate([al[:e], al[ep:ep + e]])[:, None]
    return (y, alpha)


# speedup vs baseline: 1.5707x; 1.1567x over previous
import functools

import jax
import jax.numpy as jnp
from jax import lax
from jax.experimental import pallas as pl
from jax.experimental.pallas import tpu as pltpu
from jax.experimental.pallas import tpu_sc as plsc

_N = 50000
_NP = 50048
_D = 32
_NC = 2
_NS = 16
_NW = _NC * _NS
_L = 16
_B = 128
_IC = 512
_NOUT_U = 25088
_EPS = 1e-16

_SC_PARAMS = dict(needs_layout_passes=False, use_tc_tiling_on_sc=False)


def _conv_body(ept, nout, ndir, need_ex, epb, *refs):
    if need_ex:
        (x_ref, src_ref, dst_ref, numer_ref, denom_ref, ex_ref,
         acc_n, accd, sidx, didx, xs0, xs1, xd0, xd1, m0, m1, exb, zvec, pbuf,
         sem_s0, sem_s1, sem_d0, sem_d1, sem_m0, sem_m1, sem_e) = refs
    else:
        (x_ref, src_ref, dst_ref, numer_ref, denom_ref,
         acc_n, accd, sidx, didx, xs0, xs1, xd0, xd1, m0, m1, exb, zvec, pbuf,
         sem_s0, sem_s1, sem_d0, sem_d1, sem_m0, sem_m1, sem_e) = refs
        ex_ref = None
    xs = (xs0, xs1)
    xd = (xd0, xd1)
    mm = (m0, m1)
    sem_s = (sem_s0, sem_s1)
    sem_d = (sem_d0, sem_d1)
    sem_m = (sem_m0, sem_m1)

    c = lax.axis_index("c")
    s = lax.axis_index("s")
    tid = c * _NS + s

    zero16 = jnp.zeros((_L,), jnp.float32)

    def _zz(i, _):
        zvec[pl.ds(i * _L, _L)] = zero16
        return 0
    lax.fori_loop(0, 512 // _L, _zz, 0)

    def _zm(i, _):
        xs0[i, pl.ds(0, _L)] = zero16
        xs0[i, pl.ds(_L, _L)] = zero16
        return 0
    lax.fori_loop(0, _B, _zm, 0)

    r0 = s * (_N // _NS)
    def _zs(i, _):
        pltpu.sync_copy(xs0.at[pl.ds(0, 125)],
                        acc_n.at[pl.ds(r0 + i * 125, 125)])
        return 0
    lax.fori_loop(0, (_N // _NS) // 125, _zs, 0)
    d0 = s * (_NP // _NS)
    def _zd(i, _):
        pltpu.sync_copy(zvec, accd.at[pl.ds(d0 + i * 512, 512)])
        return 0
    lax.fori_loop(0, 6, _zd, 0)
    pltpu.sync_copy(zvec.at[pl.ds(0, 56)],
                    accd.at[pl.ds(d0 + 3072, 56)])
    plsc.subcore_barrier()

    iota16 = lax.iota(jnp.int32, _L)
    lane15 = iota16 * _L + (_L - 1)
    rows_per_chunk = _IC // _B

    if ndir == 2:
        def _chunk2(ib, _):
            base_row = tid * (ept // _B) + ib * rows_per_chunk
            pltpu.sync_copy(src_ref.at[pl.ds(base_row, rows_per_chunk)], sidx)
            pltpu.sync_copy(dst_ref.at[pl.ds(base_row, rows_per_chunk)], didx)
            pend = {}
            pend[0] = (
                pltpu.async_copy(x_ref.at[sidx.at[0]], xs[0], sem_s[0]),
                pltpu.async_copy(x_ref.at[didx.at[0]], xd[0], sem_d[0]),
            )
            scat = {}
            for jb in range(rows_per_chunk):
                sl = jb % 2
                if jb + 1 < rows_per_chunk:
                    nsl = (jb + 1) % 2
                    pend[jb + 1] = (
                        pltpu.async_copy(x_ref.at[sidx.at[jb + 1]],
                                         xs[nsl], sem_s[nsl]),
                        pltpu.async_copy(x_ref.at[didx.at[jb + 1]],
                                         xd[nsl], sem_d[nsl]),
                    )
                pend[jb][0].wait()
                pend[jb][1].wait()
                if jb >= 1:
                    scat[(jb - 1, "f")].wait()
                    scat[(jb - 1, "b")].wait()

                def _group2(g, _, _jb=jb, _xs=xs[sl], _xd=xd[sl]):
                    off = g * _L
                    vs = sidx[_jb, pl.ds(off, _L)]
                    vd = didx[_jb, pl.ds(off, _L)]
                    srows = []
                    for i in range(_L):
                        e = off + i
                        s0 = _xs[e, pl.ds(0, _L)]
                        s1 = _xs[e, pl.ds(_L, _L)]
                        srows.append((s0, s1))
                        p16 = (s0 * _xd[e, pl.ds(0, _L)] +
                               s1 * _xd[e, pl.ds(_L, _L)])
                        pbuf[pl.ds(i * _L, _L)] = plsc.cumsum(p16)
                    vdot = plsc.load_gather(pbuf, [lane15])
                    ex = jnp.where(vs != vd, jnp.exp(vdot), 0.0)
                    exb[_jb, pl.ds(off, _L)] = ex
                    for i in range(_L):
                        e = off + i
                        exi = ex[i]
                        s0, s1 = srows[i]
                        m0[e, pl.ds(0, _L)] = exi * s0
                        m0[e, pl.ds(_L, _L)] = exi * s1
                        m1[e, pl.ds(0, _L)] = exi * _xd[e, pl.ds(0, _L)]
                        m1[e, pl.ds(_L, _L)] = exi * _xd[e, pl.ds(_L, _L)]
                    return 0

                lax.fori_loop(0, _B // _L, _group2, 0)
                scat[(jb, "f")] = pltpu.async_copy(
                    m0, acc_n.at[didx.at[jb]], sem_m[0], add=True)
                scat[(jb, "b")] = pltpu.async_copy(
                    m1, acc_n.at[sidx.at[jb]], sem_m[1], add=True)
                scat[(jb, "e1")] = pltpu.async_copy(
                    exb.at[jb], accd.at[didx.at[jb]], sem_e, add=True)
                scat[(jb, "e2")] = pltpu.async_copy(
                    exb.at[jb], accd.at[sidx.at[jb]], sem_e, add=True)
            scat[(rows_per_chunk - 1, "f")].wait()
            scat[(rows_per_chunk - 1, "b")].wait()
            for jb in range(rows_per_chunk):
                scat[(jb, "e1")].wait()
                scat[(jb, "e2")].wait()
            pltpu.sync_copy(exb, ex_ref.at[pl.ds(base_row, rows_per_chunk)])
            pltpu.sync_copy(exb,
                            ex_ref.at[pl.ds(epb + base_row, rows_per_chunk)])
            return 0

        lax.fori_loop(0, ept // _IC, _chunk2, 0)

    for p in range(ndir if ndir == 1 else 0):
        s_src = src_ref if p == 0 else dst_ref
        s_dst = dst_ref if p == 0 else src_ref

        def _chunk(ib, _, _p=p, _ss=s_src, _sd=s_dst):
            base_row = tid * (ept // _B) + ib * rows_per_chunk
            pltpu.sync_copy(_ss.at[pl.ds(base_row, rows_per_chunk)], sidx)
            pltpu.sync_copy(_sd.at[pl.ds(base_row, rows_per_chunk)], didx)
            pend = {}
            pend[0] = (
                pltpu.async_copy(x_ref.at[sidx.at[0]], xs[0], sem_s[0]),
                pltpu.async_copy(x_ref.at[didx.at[0]], xd[0], sem_d[0]),
            )
            scat = {}
            for jb in range(rows_per_chunk):
                sl = jb % 2
                if jb + 1 < rows_per_chunk:
                    nsl = (jb + 1) % 2
                    pend[jb + 1] = (
                        pltpu.async_copy(x_ref.at[sidx.at[jb + 1]],
                                         xs[nsl], sem_s[nsl]),
                        pltpu.async_copy(x_ref.at[didx.at[jb + 1]],
                                         xd[nsl], sem_d[nsl]),
                    )
                pend[jb][0].wait()
                pend[jb][1].wait()
                if jb >= 2:
                    scat[jb - 2].wait()
                _xs = xs[sl]
                _xd = xd[sl]
                m = mm[sl]

                def _group(g, _, _jb=jb, _xs=_xs, _xd=_xd, m=m):
                    off = g * _L
                    vs = sidx[_jb, pl.ds(off, _L)]
                    vd = didx[_jb, pl.ds(off, _L)]
                    srows = []
                    for i in range(_L):
                        e = off + i
                        s0 = _xs[e, pl.ds(0, _L)]
                        s1 = _xs[e, pl.ds(_L, _L)]
                        srows.append((s0, s1))
                        p16 = (s0 * _xd[e, pl.ds(0, _L)] +
                               s1 * _xd[e, pl.ds(_L, _L)])
                        pbuf[pl.ds(i * _L, _L)] = plsc.cumsum(p16)
                    vdot = plsc.load_gather(pbuf, [lane15])
                    ex = jnp.where(vs != vd, jnp.exp(vdot), 0.0)
                    exb[_jb, pl.ds(off, _L)] = ex
                    for i in range(_L):
                        e = off + i
                        exi = ex[i]
                        s0, s1 = srows[i]
                        m[e, pl.ds(0, _L)] = exi * s0
                        m[e, pl.ds(_L, _L)] = exi * s1
                    return 0

                lax.fori_loop(0, _B // _L, _group, 0)
                scat[jb] = pltpu.async_copy(m, acc_n.at[didx.at[jb]],
                                            sem_m[sl], add=True)
                scat[(jb, "e")] = pltpu.async_copy(
                    exb.at[jb], accd.at[didx.at[jb]], sem_e, add=True)
            scat[rows_per_chunk - 2].wait()
            scat[rows_per_chunk - 1].wait()
            for jb in range(rows_per_chunk):
                scat[(jb, "e")].wait()
            if need_ex:
                pltpu.sync_copy(
                    exb, ex_ref.at[pl.ds(_p * epb + base_row, rows_per_chunk)])
            return 0

        lax.fori_loop(0, ept // _IC, _chunk, 0)

    plsc.subcore_barrier()

    dstripe = _NP // _NS
    pltpu.sync_copy(accd.at[pl.ds(s * dstripe, dstripe)],
                    denom_ref.at[c, pl.ds(s * dstripe, dstripe)])

    rows_per_tile = nout // _NS
    chunk = 112 if rows_per_tile % 125 else 125
    q0 = s * rows_per_tile
    def _cp(i, _):
        pltpu.sync_copy(acc_n.at[pl.ds(q0 + i * chunk, chunk)],
                        m0.at[pl.ds(0, chunk)])
        pltpu.sync_copy(m0.at[pl.ds(0, chunk)],
                        numer_ref.at[c, pl.ds(q0 + i * chunk, chunk)])
        return 0
    lax.fori_loop(0, rows_per_tile // chunk, _cp, 0)


def _make_conv(ept, nout, ndir, need_ex, epb):
    mesh = plsc.VectorSubcoreMesh(core_axis_name="c", subcore_axis_name="s",
                                  num_cores=_NC, num_subcores=_NS)
    outs = [
        jax.ShapeDtypeStruct((_NC, nout, _D), jnp.float32),
        jax.ShapeDtypeStruct((_NC, _NP), jnp.float32),
    ]
    if need_ex:
        outs.append(jax.ShapeDtypeStruct((ndir * epb, _B), jnp.float32))
    scratch = [
        pltpu.VMEM_SHARED((_N, _D), jnp.float32),
        pltpu.VMEM_SHARED((_NP,), jnp.float32),
        pltpu.VMEM((_IC // _B, _B), jnp.int32),
        pltpu.VMEM((_IC // _B, _B), jnp.int32),
        pltpu.VMEM((_B, _D), jnp.float32),
        pltpu.VMEM((_B, _D), jnp.float32),
        pltpu.VMEM((_B, _D), jnp.float32),
        pltpu.VMEM((_B, _D), jnp.float32),
        pltpu.VMEM((_B, _D), jnp.float32),
        pltpu.VMEM((_B, _D), jnp.float32),
        pltpu.VMEM((_IC // _B, _B), jnp.float32),
        pltpu.VMEM((512,), jnp.float32),
        pltpu.VMEM((_B * 2,), jnp.float32),
        pltpu.SemaphoreType.DMA,
        pltpu.SemaphoreType.DMA,
        pltpu.SemaphoreType.DMA,
        pltpu.SemaphoreType.DMA,
        pltpu.SemaphoreType.DMA,
        pltpu.SemaphoreType.DMA,
        pltpu.SemaphoreType.DMA,
    ]
    body = functools.partial(_conv_body, ept, nout, ndir, need_ex, epb)
    return pl.kernel(body, out_type=tuple(outs), mesh=mesh,
                     compiler_params=pltpu.CompilerParams(**_SC_PARAMS),
                     scratch_types=scratch)


def _alpha_body(ept, epb, den_ref, src_ref, dst_ref, ex_ref, al_ref,
                denl, didx, exb, alb):
    c = lax.axis_index("c")
    s = lax.axis_index("s")
    tid = c * _NS + s
    pltpu.sync_copy(den_ref, denl)
    rows_per_chunk = _IC // _B
    for p in range(2):
        dref = dst_ref if p == 0 else src_ref

        def _chunk(ib, _, _p=p, _dref=dref):
            base = tid * (ept // _B) + ib * rows_per_chunk
            pltpu.sync_copy(_dref.at[pl.ds(base, rows_per_chunk)], didx)
            pltpu.sync_copy(ex_ref.at[pl.ds(_p * epb + base, rows_per_chunk)],
                            exb)
            for jb in range(rows_per_chunk):
                def _grp(g, _, _jb=jb):
                    off = g * _L
                    vd = didx[_jb, pl.ds(off, _L)]
                    dv = plsc.load_gather(denl, [vd])
                    ev = exb[_jb, pl.ds(off, _L)]
                    alb[_jb, pl.ds(off, _L)] = ev / (dv + _EPS)
                    return 0
                lax.fori_loop(0, _B // _L, _grp, 0)
            pltpu.sync_copy(alb, al_ref.at[pl.ds(_p * epb + base,
                                                 rows_per_chunk)])
            return 0

        lax.fori_loop(0, ept // _IC, _chunk, 0)


def _make_alpha(ept, epb):
    mesh = plsc.VectorSubcoreMesh(core_axis_name="c", subcore_axis_name="s",
                                  num_cores=_NC, num_subcores=_NS)
    scratch = [
        pltpu.VMEM((_N,), jnp.float32),
        pltpu.VMEM((_IC // _B, _B), jnp.int32),
        pltpu.VMEM((_IC // _B, _B), jnp.float32),
        pltpu.VMEM((_IC // _B, _B), jnp.float32),
    ]
    body = functools.partial(_alpha_body, ept, epb)
    return pl.kernel(body,
                     out_type=jax.ShapeDtypeStruct((2 * epb, _B), jnp.float32),
                     mesh=mesh,
                     compiler_params=pltpu.CompilerParams(**_SC_PARAMS),
                     scratch_types=scratch)


_RB = 1000


def _prep_feats_body(f_ref, w_ref, b_ref, o_ref):
    y = lax.dot_general(f_ref[...], w_ref[...],
                        (((1,), (1,)), ((), ())),
                        preferred_element_type=jnp.float32)
    y = y + b_ref[...]
    y = jnp.where(y > 0, y, 0.01 * y)
    n = jnp.sqrt(jnp.sum(y * y, axis=1, keepdims=True))
    o_ref[...] = y / jnp.maximum(n, 1e-12)


def _norm_body(p_ref, o_ref):
    p = p_ref[...]
    n = jnp.sqrt(jnp.sum(p * p, axis=1, keepdims=True))
    o_ref[...] = p / jnp.maximum(n, 1e-12)


def _route_body(p_ref, n_ref, d_ref, o_ref):
    num = n_ref[0] + n_ref[1]
    den = d_ref[0, 0, 0, :] + d_ref[1, 0, 0, :]
    out = num / (den[:, None] + _EPS)
    p2 = p_ref[...] + out
    nn = jnp.sqrt(jnp.sum(p2 * p2, axis=1, keepdims=True))
    o_ref[...] = p2 / jnp.maximum(nn, 1e-12)


def _final_body(x_ref, n_ref, d_ref, y_ref, dt_ref):
    num = n_ref[0] + n_ref[1]
    den = d_ref[0, 0, 0, :] + d_ref[1, 0, 0, :]
    out = num / (den[:, None] + _EPS)
    out = jnp.where(out > 0, out, 0.01 * out)
    y_ref[...] = x_ref[...] + out
    dt_ref[...] = jnp.broadcast_to(den[None, None, None, :],
                                   (8, 1, 1, den.shape[0]))


def kernel(feature, edge_index, preference, W, b):
    nu = preference.shape[0]
    ni = feature.shape[0]
    n_nodes = nu + ni
    e = edge_index.shape[1]

    ep = -(-e // (_NW * _IC)) * (_NW * _IC)
    epb = ep // _B
    ept = ep // _NW
    pad = ep - e
    src = jnp.concatenate([edge_index[0], jnp.zeros((pad,), jnp.int32)])
    dst = jnp.concatenate([edge_index[1], jnp.zeros((pad,), jnp.int32)])
    src2d = src.reshape(epb, _B)
    dst2d = dst.reshape(epb, _B)

    feats = pl.pallas_call(
        _prep_feats_body,
        grid=(ni // _RB,),
        in_specs=[
            pl.BlockSpec((_RB, feature.shape[1]), lambda i: (i, 0)),
            pl.BlockSpec(W.shape, lambda i: (0, 0)),
            pl.BlockSpec((1, _D), lambda i: (0, 0)),
        ],
        out_specs=pl.BlockSpec((_RB, _D), lambda i: (i, 0)),
        out_shape=jax.ShapeDtypeStruct((ni, _D), jnp.float32),
    )(feature, W, b.reshape(1, _D))

    pref = pl.pallas_call(
        _norm_body,
        grid=(nu // _RB,),
        in_specs=[pl.BlockSpec((_RB, _D), lambda i: (i, 0))],
        out_specs=pl.BlockSpec((_RB, _D), lambda i: (i, 0)),
        out_shape=jax.ShapeDtypeStruct((nu, _D), jnp.float32),
    )(preference)

    conv_route = _make_conv(ept, _NOUT_U, 1, False, epb)
    route = pl.pallas_call(
        _route_body,
        grid=(nu // _RB,),
        in_specs=[
            pl.BlockSpec((_RB, _D), lambda i: (i, 0)),
            pl.BlockSpec((_NC, _RB, _D), lambda i: (0, i, 0)),
            pl.BlockSpec((_NC, 1, 1, _RB), lambda i: (0, i, 0, 0)),
        ],
        out_specs=pl.BlockSpec((_RB, _D), lambda i: (i, 0)),
        out_shape=jax.ShapeDtypeStruct((nu, _D), jnp.float32),
    )

    for _ in range(2):
        x = jnp.concatenate([pref, feats], axis=0)
        numer_p, denom_p = conv_route(x, src2d, dst2d)
        d3 = denom_p[:, :nu].reshape(_NC, nu // _RB, 1, _RB)
        pref = route(pref, numer_p[:, :nu], d3)

    x = jnp.concatenate([pref, feats], axis=0)
    conv_final = _make_conv(ept, n_nodes, 2, True, epb)
    numer_p, denom_p, ex2d = conv_final(x, src2d, dst2d)

    y, dt = pl.pallas_call(
        _final_body,
        grid=(n_nodes // _RB,),
        in_specs=[
            pl.BlockSpec((_RB, _D), lambda i: (i, 0)),
            pl.BlockSpec((_NC, _RB, _D), lambda i: (0, i, 0)),
            pl.BlockSpec((_NC, 1, 1, _RB), lambda i: (0, i, 0, 0)),
        ],
        out_specs=[
            pl.BlockSpec((_RB, _D), lambda i: (i, 0)),
            pl.BlockSpec((8, 1, 1, _RB), lambda i: (0, i, 0, 0)),
        ],
        out_shape=[
            jax.ShapeDtypeStruct((n_nodes, _D), jnp.float32),
            jax.ShapeDtypeStruct((8, n_nodes // _RB, 1, _RB), jnp.float32),
        ],
    )(x, numer_p, denom_p[:, :n_nodes].reshape(_NC, n_nodes // _RB, 1, _RB))

    al2d = _make_alpha(ept, epb)(dt[0].reshape(n_nodes), src2d, dst2d, ex2d)
    al = al2d.reshape(-1)
    alpha = jnp.concatenate([al[:e], al[ep:ep + e]])[:, None]
    return (y, alpha)


# speedup vs baseline: 1.6090x; 1.0244x over previous
import functools

import jax
import jax.numpy as jnp
from jax import lax
from jax.experimental import pallas as pl
from jax.experimental.pallas import tpu as pltpu
from jax.experimental.pallas import tpu_sc as plsc

_N = 50000
_NP = 50048
_D = 32
_NC = 2
_NS = 16
_NW = _NC * _NS
_L = 16
_B = 128
_IC = 512
_NOUT_U = 25088
_EPS = 1e-16

_SC_PARAMS = dict(needs_layout_passes=False, use_tc_tiling_on_sc=False)


def _conv_body(ept, nout, ndir, need_ex, epb, *refs):
    if need_ex:
        (x_ref, src_ref, dst_ref, numer_ref, denom_ref, ex_ref,
         acc_n, accd, sidx, didx, xs0, xs1, xd0, xd1, m0, m1, exb, zvec, pbuf,
         sem_s0, sem_s1, sem_d0, sem_d1, sem_m0, sem_m1, sem_e) = refs
    else:
        (x_ref, src_ref, dst_ref, numer_ref, denom_ref,
         acc_n, accd, sidx, didx, xs0, xs1, xd0, xd1, m0, m1, exb, zvec, pbuf,
         sem_s0, sem_s1, sem_d0, sem_d1, sem_m0, sem_m1, sem_e) = refs
        ex_ref = None
    xs = (xs0, xs1)
    xd = (xd0, xd1)
    mm = (m0, m1)
    sem_s = (sem_s0, sem_s1)
    sem_d = (sem_d0, sem_d1)
    sem_m = (sem_m0, sem_m1)

    c = lax.axis_index("c")
    s = lax.axis_index("s")
    tid = c * _NS + s

    zero16 = jnp.zeros((_L,), jnp.float32)

    def _zz(i, _):
        zvec[pl.ds(i * _L, _L)] = zero16
        return 0
    lax.fori_loop(0, 512 // _L, _zz, 0)

    def _zm(i, _):
        xs0[i, pl.ds(0, _L)] = zero16
        xs0[i, pl.ds(_L, _L)] = zero16
        return 0
    lax.fori_loop(0, _B, _zm, 0)

    r0 = s * (_N // _NS)
    def _zs(i, _):
        pltpu.sync_copy(xs0.at[pl.ds(0, 125)],
                        acc_n.at[pl.ds(r0 + i * 125, 125)])
        return 0
    lax.fori_loop(0, (_N // _NS) // 125, _zs, 0)
    d0 = s * (_NP // _NS)
    def _zd(i, _):
        pltpu.sync_copy(zvec, accd.at[pl.ds(d0 + i * 512, 512)])
        return 0
    lax.fori_loop(0, 6, _zd, 0)
    pltpu.sync_copy(zvec.at[pl.ds(0, 56)],
                    accd.at[pl.ds(d0 + 3072, 56)])
    plsc.subcore_barrier()

    iota16 = lax.iota(jnp.int32, _L)
    lane15 = iota16 * _L + (_L - 1)
    rows_per_chunk = _IC // _B

    if ndir == 2:
        def _chunk2(ib, _):
            base_row = tid * (ept // _B) + ib * rows_per_chunk
            pltpu.sync_copy(src_ref.at[pl.ds(base_row, rows_per_chunk)], sidx)
            pltpu.sync_copy(dst_ref.at[pl.ds(base_row, rows_per_chunk)], didx)
            pend = {}
            pend[0] = (
                pltpu.async_copy(x_ref.at[sidx.at[0]], xs[0], sem_s[0]),
                pltpu.async_copy(x_ref.at[didx.at[0]], xd[0], sem_d[0]),
            )
            scat = {}
            for jb in range(rows_per_chunk):
                sl = jb % 2
                if jb + 1 < rows_per_chunk:
                    nsl = (jb + 1) % 2
                    pend[jb + 1] = (
                        pltpu.async_copy(x_ref.at[sidx.at[jb + 1]],
                                         xs[nsl], sem_s[nsl]),
                        pltpu.async_copy(x_ref.at[didx.at[jb + 1]],
                                         xd[nsl], sem_d[nsl]),
                    )
                pend[jb][0].wait()
                pend[jb][1].wait()
                if jb >= 1:
                    scat[(jb - 1, "f")].wait()
                    scat[(jb - 1, "b")].wait()

                def _group2(g, _, _jb=jb, _xs=xs[sl], _xd=xd[sl]):
                    off = g * _L
                    vs = sidx[_jb, pl.ds(off, _L)]
                    vd = didx[_jb, pl.ds(off, _L)]
                    srows = []
                    for i in range(_L):
                        e = off + i
                        s0 = _xs[e, pl.ds(0, _L)]
                        s1 = _xs[e, pl.ds(_L, _L)]
                        srows.append((s0, s1))
                        p16 = (s0 * _xd[e, pl.ds(0, _L)] +
                               s1 * _xd[e, pl.ds(_L, _L)])
                        pbuf[pl.ds(i * _L, _L)] = plsc.cumsum(p16)
                    vdot = plsc.load_gather(pbuf, [lane15])
                    ex = jnp.where(vs != vd, jnp.exp(vdot), 0.0)
                    exb[_jb, pl.ds(off, _L)] = ex
                    for i in range(_L):
                        e = off + i
                        exi = ex[i]
                        s0, s1 = srows[i]
                        m0[e, pl.ds(0, _L)] = exi * s0
                        m0[e, pl.ds(_L, _L)] = exi * s1
                        m1[e, pl.ds(0, _L)] = exi * _xd[e, pl.ds(0, _L)]
                        m1[e, pl.ds(_L, _L)] = exi * _xd[e, pl.ds(_L, _L)]
                    return 0

                lax.fori_loop(0, _B // _L, _group2, 0)
                scat[(jb, "f")] = pltpu.async_copy(
                    m0, acc_n.at[didx.at[jb]], sem_m[0], add=True)
                scat[(jb, "b")] = pltpu.async_copy(
                    m1, acc_n.at[sidx.at[jb]], sem_m[1], add=True)
                scat[(jb, "e1")] = pltpu.async_copy(
                    exb.at[jb], accd.at[didx.at[jb]], sem_e, add=True)
                scat[(jb, "e2")] = pltpu.async_copy(
                    exb.at[jb], accd.at[sidx.at[jb]], sem_e, add=True)
            scat[(rows_per_chunk - 1, "f")].wait()
            scat[(rows_per_chunk - 1, "b")].wait()
            for jb in range(rows_per_chunk):
                scat[(jb, "e1")].wait()
                scat[(jb, "e2")].wait()
            pltpu.sync_copy(exb, ex_ref.at[pl.ds(base_row, rows_per_chunk)])
            pltpu.sync_copy(exb,
                            ex_ref.at[pl.ds(epb + base_row, rows_per_chunk)])
            return 0

        lax.fori_loop(0, ept // _IC, _chunk2, 0)

    for p in range(ndir if ndir == 1 else 0):
        s_src = src_ref if p == 0 else dst_ref
        s_dst = dst_ref if p == 0 else src_ref

        def _chunk(ib, _, _p=p, _ss=s_src, _sd=s_dst):
            base_row = tid * (ept // _B) + ib * rows_per_chunk
            pltpu.sync_copy(_ss.at[pl.ds(base_row, rows_per_chunk)], sidx)
            pltpu.sync_copy(_sd.at[pl.ds(base_row, rows_per_chunk)], didx)
            pend = {}
            pend[0] = (
                pltpu.async_copy(x_ref.at[sidx.at[0]], xs[0], sem_s[0]),
                pltpu.async_copy(x_ref.at[didx.at[0]], xd[0], sem_d[0]),
            )
            scat = {}
            for jb in range(rows_per_chunk):
                sl = jb % 2
                if jb + 1 < rows_per_chunk:
                    nsl = (jb + 1) % 2
                    pend[jb + 1] = (
                        pltpu.async_copy(x_ref.at[sidx.at[jb + 1]],
                                         xs[nsl], sem_s[nsl]),
                        pltpu.async_copy(x_ref.at[didx.at[jb + 1]],
                                         xd[nsl], sem_d[nsl]),
                    )
                pend[jb][0].wait()
                pend[jb][1].wait()
                if jb >= 2:
                    scat[jb - 2].wait()
                _xs = xs[sl]
                _xd = xd[sl]
                m = mm[sl]

                def _group(g, _, _jb=jb, _xs=_xs, _xd=_xd, m=m):
                    off = g * _L
                    vs = sidx[_jb, pl.ds(off, _L)]
                    vd = didx[_jb, pl.ds(off, _L)]
                    srows = []
                    for i in range(_L):
                        e = off + i
                        s0 = _xs[e, pl.ds(0, _L)]
                        s1 = _xs[e, pl.ds(_L, _L)]
                        srows.append((s0, s1))
                        p16 = (s0 * _xd[e, pl.ds(0, _L)] +
                               s1 * _xd[e, pl.ds(_L, _L)])
                        pbuf[pl.ds(i * _L, _L)] = plsc.cumsum(p16)
                    vdot = plsc.load_gather(pbuf, [lane15])
                    ex = jnp.where(vs != vd, jnp.exp(vdot), 0.0)
                    exb[_jb, pl.ds(off, _L)] = ex
                    for i in range(_L):
                        e = off + i
                        exi = ex[i]
                        s0, s1 = srows[i]
                        m[e, pl.ds(0, _L)] = exi * s0
                        m[e, pl.ds(_L, _L)] = exi * s1
                    return 0

                lax.fori_loop(0, _B // _L, _group, 0)
                scat[jb] = pltpu.async_copy(m, acc_n.at[didx.at[jb]],
                                            sem_m[sl], add=True)
                scat[(jb, "e")] = pltpu.async_copy(
                    exb.at[jb], accd.at[didx.at[jb]], sem_e, add=True)
            scat[rows_per_chunk - 2].wait()
            scat[rows_per_chunk - 1].wait()
            for jb in range(rows_per_chunk):
                scat[(jb, "e")].wait()
            if need_ex:
                pltpu.sync_copy(
                    exb, ex_ref.at[pl.ds(_p * epb + base_row, rows_per_chunk)])
            return 0

        lax.fori_loop(0, ept // _IC, _chunk, 0)

    plsc.subcore_barrier()

    dstripe = _NP // _NS
    pltpu.sync_copy(accd.at[pl.ds(s * dstripe, dstripe)],
                    denom_ref.at[c, pl.ds(s * dstripe, dstripe)])

    rows_per_tile = nout // _NS
    chunk = 112 if rows_per_tile % 125 else 125
    q0 = s * rows_per_tile
    def _cp(i, _):
        pltpu.sync_copy(acc_n.at[pl.ds(q0 + i * chunk, chunk)],
                        m0.at[pl.ds(0, chunk)])
        pltpu.sync_copy(m0.at[pl.ds(0, chunk)],
                        numer_ref.at[c, pl.ds(q0 + i * chunk, chunk)])
        return 0
    lax.fori_loop(0, rows_per_tile // chunk, _cp, 0)


def _make_conv(ept, nout, ndir, need_ex, epb):
    mesh = plsc.VectorSubcoreMesh(core_axis_name="c", subcore_axis_name="s",
                                  num_cores=_NC, num_subcores=_NS)
    outs = [
        jax.ShapeDtypeStruct((_NC, nout, _D), jnp.float32),
        jax.ShapeDtypeStruct((_NC, _NP), jnp.float32),
    ]
    if need_ex:
        outs.append(jax.ShapeDtypeStruct((ndir * epb, _B), jnp.float32))
    scratch = [
        pltpu.VMEM_SHARED((_N, _D), jnp.float32),
        pltpu.VMEM_SHARED((_NP,), jnp.float32),
        pltpu.VMEM((_IC // _B, _B), jnp.int32),
        pltpu.VMEM((_IC // _B, _B), jnp.int32),
        pltpu.VMEM((_B, _D), jnp.float32),
        pltpu.VMEM((_B, _D), jnp.float32),
        pltpu.VMEM((_B, _D), jnp.float32),
        pltpu.VMEM((_B, _D), jnp.float32),
        pltpu.VMEM((_B, _D), jnp.float32),
        pltpu.VMEM((_B, _D), jnp.float32),
        pltpu.VMEM((_IC // _B, _B), jnp.float32),
        pltpu.VMEM((512,), jnp.float32),
        pltpu.VMEM((_B * 2,), jnp.float32),
        pltpu.SemaphoreType.DMA,
        pltpu.SemaphoreType.DMA,
        pltpu.SemaphoreType.DMA,
        pltpu.SemaphoreType.DMA,
        pltpu.SemaphoreType.DMA,
        pltpu.SemaphoreType.DMA,
        pltpu.SemaphoreType.DMA,
    ]
    body = functools.partial(_conv_body, ept, nout, ndir, need_ex, epb)
    return pl.kernel(body, out_type=tuple(outs), mesh=mesh,
                     compiler_params=pltpu.CompilerParams(**_SC_PARAMS),
                     scratch_types=scratch)


def _alpha_body(ept, epb, den_ref, src_ref, dst_ref, ex_ref, al_ref,
                denl, didx0, didx1, exb0, exb1, alb0, alb1,
                sem_i0, sem_i1, sem_e0, sem_e1, sem_o0, sem_o1):
    c = lax.axis_index("c")
    s = lax.axis_index("s")
    tid = c * _NS + s
    pltpu.sync_copy(den_ref, denl)
    rows_per_chunk = _IC // _B
    didx = (didx0, didx1)
    exb = (exb0, exb1)
    alb = (alb0, alb1)
    sem_i = (sem_i0, sem_i1)
    sem_e = (sem_e0, sem_e1)
    sem_o = (sem_o0, sem_o1)
    nchunk = ept // _IC

    def _do(base, _p, sl):
        for jb in range(rows_per_chunk):
            def _grp(g, _, _jb=jb, _sl=sl):
                off = g * _L
                vd = didx[_sl][_jb, pl.ds(off, _L)]
                dv = plsc.load_gather(denl, [vd])
                ev = exb[_sl][_jb, pl.ds(off, _L)]
                alb[_sl][_jb, pl.ds(off, _L)] = ev / (dv + _EPS)
                return 0
            lax.fori_loop(0, _B // _L, _grp, 0)
        return pltpu.async_copy(
            alb[sl], al_ref.at[pl.ds(_p * epb + base, rows_per_chunk)],
            sem_o[sl])

    for p in range(2):
        dref = dst_ref if p == 0 else src_ref

        def _pair(q, _, _p=p, _dref=dref):
            baseA = tid * (ept // _B) + (2 * q) * rows_per_chunk
            baseB = baseA + rows_per_chunk
            ldA = (pltpu.async_copy(_dref.at[pl.ds(baseA, rows_per_chunk)],
                                    didx[0], sem_i[0]),
                   pltpu.async_copy(
                       ex_ref.at[pl.ds(_p * epb + baseA, rows_per_chunk)],
                       exb[0], sem_e[0]))
            ldB = (pltpu.async_copy(_dref.at[pl.ds(baseB, rows_per_chunk)],
                                    didx[1], sem_i[1]),
                   pltpu.async_copy(
                       ex_ref.at[pl.ds(_p * epb + baseB, rows_per_chunk)],
                       exb[1], sem_e[1]))
            ldA[0].wait(); ldA[1].wait()
            stA = _do(baseA, _p, 0)
            ldB[0].wait(); ldB[1].wait()
            stB = _do(baseB, _p, 1)
            stA.wait(); stB.wait()
            return 0

        lax.fori_loop(0, nchunk // 2, _pair, 0)
        if nchunk % 2:
            base = tid * (ept // _B) + (nchunk - 1) * rows_per_chunk
            pltpu.async_copy(dref.at[pl.ds(base, rows_per_chunk)],
                             didx[0], sem_i[0]).wait()
            pltpu.async_copy(ex_ref.at[pl.ds(p * epb + base, rows_per_chunk)],
                             exb[0], sem_e[0]).wait()
            _do(base, p, 0).wait()


def _make_alpha(ept, epb):
    mesh = plsc.VectorSubcoreMesh(core_axis_name="c", subcore_axis_name="s",
                                  num_cores=_NC, num_subcores=_NS)
    scratch = [
        pltpu.VMEM((_N,), jnp.float32),
        pltpu.VMEM((_IC // _B, _B), jnp.int32),
        pltpu.VMEM((_IC // _B, _B), jnp.int32),
        pltpu.VMEM((_IC // _B, _B), jnp.float32),
        pltpu.VMEM((_IC // _B, _B), jnp.float32),
        pltpu.VMEM((_IC // _B, _B), jnp.float32),
        pltpu.VMEM((_IC // _B, _B), jnp.float32),
    ] + [pltpu.SemaphoreType.DMA] * 6
    body = functools.partial(_alpha_body, ept, epb)
    return pl.kernel(body,
                     out_type=jax.ShapeDtypeStruct((2 * epb, _B), jnp.float32),
                     mesh=mesh,
                     compiler_params=pltpu.CompilerParams(**_SC_PARAMS),
                     scratch_types=scratch)


_RB = 1000


def _prep_feats_body(f_ref, w_ref, b_ref, o_ref):
    y = lax.dot_general(f_ref[...], w_ref[...],
                        (((1,), (1,)), ((), ())),
                        preferred_element_type=jnp.float32)
    y = y + b_ref[...]
    y = jnp.where(y > 0, y, 0.01 * y)
    n = jnp.sqrt(jnp.sum(y * y, axis=1, keepdims=True))
    o_ref[...] = y / jnp.maximum(n, 1e-12)


def _norm_body(p_ref, o_ref):
    p = p_ref[...]
    n = jnp.sqrt(jnp.sum(p * p, axis=1, keepdims=True))
    o_ref[...] = p / jnp.maximum(n, 1e-12)


def _route_body(p_ref, n_ref, d_ref, o_ref):
    num = n_ref[0] + n_ref[1]
    den = d_ref[0, 0, 0, :] + d_ref[1, 0, 0, :]
    out = num / (den[:, None] + _EPS)
    p2 = p_ref[...] + out
    nn = jnp.sqrt(jnp.sum(p2 * p2, axis=1, keepdims=True))
    o_ref[...] = p2 / jnp.maximum(nn, 1e-12)


def _final_body(x_ref, n_ref, d_ref, y_ref, dt_ref):
    num = n_ref[0] + n_ref[1]
    den = d_ref[0, 0, 0, :] + d_ref[1, 0, 0, :]
    out = num / (den[:, None] + _EPS)
    out = jnp.where(out > 0, out, 0.01 * out)
    y_ref[...] = x_ref[...] + out
    dt_ref[...] = jnp.broadcast_to(den[None, None, None, :],
                                   (8, 1, 1, den.shape[0]))


def kernel(feature, edge_index, preference, W, b):
    nu = preference.shape[0]
    ni = feature.shape[0]
    n_nodes = nu + ni
    e = edge_index.shape[1]

    ep = -(-e // (_NW * _IC)) * (_NW * _IC)
    epb = ep // _B
    ept = ep // _NW
    pad = ep - e
    src = jnp.concatenate([edge_index[0], jnp.zeros((pad,), jnp.int32)])
    dst = jnp.concatenate([edge_index[1], jnp.zeros((pad,), jnp.int32)])
    src2d = src.reshape(epb, _B)
    dst2d = dst.reshape(epb, _B)

    feats = pl.pallas_call(
        _prep_feats_body,
        grid=(ni // _RB,),
        in_specs=[
            pl.BlockSpec((_RB, feature.shape[1]), lambda i: (i, 0)),
            pl.BlockSpec(W.shape, lambda i: (0, 0)),
            pl.BlockSpec((1, _D), lambda i: (0, 0)),
        ],
        out_specs=pl.BlockSpec((_RB, _D), lambda i: (i, 0)),
        out_shape=jax.ShapeDtypeStruct((ni, _D), jnp.float32),
    )(feature, W, b.reshape(1, _D))

    pref = pl.pallas_call(
        _norm_body,
        grid=(nu // _RB,),
        in_specs=[pl.BlockSpec((_RB, _D), lambda i: (i, 0))],
        out_specs=pl.BlockSpec((_RB, _D), lambda i: (i, 0)),
        out_shape=jax.ShapeDtypeStruct((nu, _D), jnp.float32),
    )(preference)

    conv_route = _make_conv(ept, _NOUT_U, 1, False, epb)
    route = pl.pallas_call(
        _route_body,
        grid=(nu // _RB,),
        in_specs=[
            pl.BlockSpec((_RB, _D), lambda i: (i, 0)),
            pl.BlockSpec((_NC, _RB, _D), lambda i: (0, i, 0)),
            pl.BlockSpec((_NC, 1, 1, _RB), lambda i: (0, i, 0, 0)),
        ],
        out_specs=pl.BlockSpec((_RB, _D), lambda i: (i, 0)),
        out_shape=jax.ShapeDtypeStruct((nu, _D), jnp.float32),
    )

    for _ in range(2):
        x = jnp.concatenate([pref, feats], axis=0)
        numer_p, denom_p = conv_route(x, src2d, dst2d)
        d3 = denom_p[:, :nu].reshape(_NC, nu // _RB, 1, _RB)
        pref = route(pref, numer_p[:, :nu], d3)

    x = jnp.concatenate([pref, feats], axis=0)
    conv_final = _make_conv(ept, n_nodes, 2, True, epb)
    numer_p, denom_p, ex2d = conv_final(x, src2d, dst2d)

    y, dt = pl.pallas_call(
        _final_body,
        grid=(n_nodes // _RB,),
        in_specs=[
            pl.BlockSpec((_RB, _D), lambda i: (i, 0)),
            pl.BlockSpec((_NC, _RB, _D), lambda i: (0, i, 0)),
            pl.BlockSpec((_NC, 1, 1, _RB), lambda i: (0, i, 0, 0)),
        ],
        out_specs=[
            pl.BlockSpec((_RB, _D), lambda i: (i, 0)),
            pl.BlockSpec((8, 1, 1, _RB), lambda i: (0, i, 0, 0)),
        ],
        out_shape=[
            jax.ShapeDtypeStruct((n_nodes, _D), jnp.float32),
            jax.ShapeDtypeStruct((8, n_nodes // _RB, 1, _RB), jnp.float32),
        ],
    )(x, numer_p, denom_p[:, :n_nodes].reshape(_NC, n_nodes // _RB, 1, _RB))

    al2d = _make_alpha(ept, epb)(dt[0].reshape(n_nodes), src2d, dst2d, ex2d)
    al = al2d.reshape(-1)
    alpha = jnp.concatenate([al[:e], al[ep:ep + e]])[:, None]
    return (y, alpha)


# speedup vs baseline: 1.7574x; 1.0922x over previous
import functools

import jax
import jax.numpy as jnp
from jax import lax
from jax.experimental import pallas as pl
from jax.experimental.pallas import tpu as pltpu
from jax.experimental.pallas import tpu_sc as plsc

_N = 50000
_NP = 50048
_D = 32
_NC = 2
_NS = 16
_NW = _NC * _NS
_L = 16
_B = 128
_IC = 896
_NOUT_U = 25088
_EPS = 1e-16

_SC_PARAMS = dict(needs_layout_passes=False, use_tc_tiling_on_sc=False)


def _conv_body(ept, nout, ndir, need_ex, epb, *refs):
    if need_ex:
        (x_ref, src_ref, dst_ref, numer_ref, denom_ref, ex_ref,
         acc_n, accd, sidx, didx, xs0, xs1, xd0, xd1, m0, m1, exb, pbuf,
         sem_s0, sem_s1, sem_d0, sem_d1, sem_m0, sem_m1, sem_e,
         sem_i0, sem_i1) = refs
    else:
        (x_ref, src_ref, dst_ref, numer_ref, denom_ref,
         acc_n, accd, sidx, didx, xs0, xs1, xd0, xd1, m0, m1, exb, pbuf,
         sem_s0, sem_s1, sem_d0, sem_d1, sem_m0, sem_m1, sem_e,
         sem_i0, sem_i1) = refs
        ex_ref = None
    xs = (xs0, xs1)
    xd = (xd0, xd1)
    mm = (m0, m1)
    sem_s = (sem_s0, sem_s1)
    sem_d = (sem_d0, sem_d1)
    sem_m = (sem_m0, sem_m1)

    c = lax.axis_index("c")
    s = lax.axis_index("s")
    tid = c * _NS + s

    zero16 = jnp.zeros((_L,), jnp.float32)

    def _zz(i, _):
        pbuf[pl.ds(i * _L, _L)] = zero16
        return 0
    lax.fori_loop(0, 256 // _L, _zz, 0)

    def _zm(i, _):
        xs0[i, pl.ds(0, _L)] = zero16
        xs0[i, pl.ds(_L, _L)] = zero16
        return 0
    lax.fori_loop(0, _B, _zm, 0)

    r0 = s * (_N // _NS)
    def _zs(i, _):
        pltpu.sync_copy(xs0.at[pl.ds(0, 125)],
                        acc_n.at[pl.ds(r0 + i * 125, 125)])
        return 0
    lax.fori_loop(0, (_N // _NS) // 125, _zs, 0)
    d0 = s * (_NP // _NS)
    def _zd(i, _):
        pltpu.sync_copy(pbuf, accd.at[pl.ds(d0 + i * 256, 256)])
        return 0
    lax.fori_loop(0, 12, _zd, 0)
    pltpu.sync_copy(pbuf.at[pl.ds(0, 56)],
                    accd.at[pl.ds(d0 + 3072, 56)])
    plsc.subcore_barrier()

    iota16 = lax.iota(jnp.int32, _L)
    lane15 = iota16 * _L + (_L - 1)
    rows_per_chunk = _IC // _B

    if ndir == 2:
        def _chunk2(ib, _):
            base_row = tid * (ept // _B) + ib * rows_per_chunk
            ld0 = pltpu.async_copy(
                src_ref.at[pl.ds(base_row, rows_per_chunk)], sidx, sem_i0)
            ld1 = pltpu.async_copy(
                dst_ref.at[pl.ds(base_row, rows_per_chunk)], didx, sem_i1)
            ld0.wait()
            ld1.wait()
            pend = {}
            pend[0] = (
                pltpu.async_copy(x_ref.at[sidx.at[0]], xs[0], sem_s[0]),
                pltpu.async_copy(x_ref.at[didx.at[0]], xd[0], sem_d[0]),
            )
            scat = {}
            for jb in range(rows_per_chunk):
                sl = jb % 2
                if jb + 1 < rows_per_chunk:
                    nsl = (jb + 1) % 2
                    pend[jb + 1] = (
                        pltpu.async_copy(x_ref.at[sidx.at[jb + 1]],
                                         xs[nsl], sem_s[nsl]),
                        pltpu.async_copy(x_ref.at[didx.at[jb + 1]],
                                         xd[nsl], sem_d[nsl]),
                    )
                pend[jb][0].wait()
                pend[jb][1].wait()
                if jb >= 1:
                    scat[(jb - 1, "f")].wait()
                    scat[(jb - 1, "b")].wait()

                def _group2(g, _, _jb=jb, _xs=xs[sl], _xd=xd[sl]):
                    off = g * _L
                    vs = sidx[_jb, pl.ds(off, _L)]
                    vd = didx[_jb, pl.ds(off, _L)]
                    srows = []
                    for i in range(_L):
                        e = off + i
                        s0 = _xs[e, pl.ds(0, _L)]
                        s1 = _xs[e, pl.ds(_L, _L)]
                        srows.append((s0, s1))
                        p16 = (s0 * _xd[e, pl.ds(0, _L)] +
                               s1 * _xd[e, pl.ds(_L, _L)])
                        pbuf[pl.ds(i * _L, _L)] = plsc.cumsum(p16)
                    vdot = plsc.load_gather(pbuf, [lane15])
                    ex = jnp.where(vs != vd, jnp.exp(vdot), 0.0)
                    exb[_jb, pl.ds(off, _L)] = ex
                    for i in range(_L):
                        e = off + i
                        exi = ex[i]
                        s0, s1 = srows[i]
                        m0[e, pl.ds(0, _L)] = exi * s0
                        m0[e, pl.ds(_L, _L)] = exi * s1
                        m1[e, pl.ds(0, _L)] = exi * _xd[e, pl.ds(0, _L)]
                        m1[e, pl.ds(_L, _L)] = exi * _xd[e, pl.ds(_L, _L)]
                    return 0

                lax.fori_loop(0, _B // _L, _group2, 0)
                scat[(jb, "f")] = pltpu.async_copy(
                    m0, acc_n.at[didx.at[jb]], sem_m[0], add=True)
                scat[(jb, "b")] = pltpu.async_copy(
                    m1, acc_n.at[sidx.at[jb]], sem_m[1], add=True)
                scat[(jb, "e1")] = pltpu.async_copy(
                    exb.at[jb], accd.at[didx.at[jb]], sem_e, add=True)
                scat[(jb, "e2")] = pltpu.async_copy(
                    exb.at[jb], accd.at[sidx.at[jb]], sem_e, add=True)
            scat[(rows_per_chunk - 1, "f")].wait()
            scat[(rows_per_chunk - 1, "b")].wait()
            for jb in range(rows_per_chunk):
                scat[(jb, "e1")].wait()
                scat[(jb, "e2")].wait()
            pltpu.sync_copy(exb, ex_ref.at[pl.ds(base_row, rows_per_chunk)])
            pltpu.sync_copy(exb,
                            ex_ref.at[pl.ds(epb + base_row, rows_per_chunk)])
            return 0

        lax.fori_loop(0, ept // _IC, _chunk2, 0)

    for p in range(ndir if ndir == 1 else 0):
        s_src = src_ref if p == 0 else dst_ref
        s_dst = dst_ref if p == 0 else src_ref

        def _chunk(ib, _, _p=p, _ss=s_src, _sd=s_dst):
            base_row = tid * (ept // _B) + ib * rows_per_chunk
            ld0 = pltpu.async_copy(
                _ss.at[pl.ds(base_row, rows_per_chunk)], sidx, sem_i0)
            ld1 = pltpu.async_copy(
                _sd.at[pl.ds(base_row, rows_per_chunk)], didx, sem_i1)
            ld0.wait()
            ld1.wait()
            pend = {}
            pend[0] = (
                pltpu.async_copy(x_ref.at[sidx.at[0]], xs[0], sem_s[0]),
                pltpu.async_copy(x_ref.at[didx.at[0]], xd[0], sem_d[0]),
            )
            scat = {}
            for jb in range(rows_per_chunk):
                sl = jb % 2
                if jb + 1 < rows_per_chunk:
                    nsl = (jb + 1) % 2
                    pend[jb + 1] = (
                        pltpu.async_copy(x_ref.at[sidx.at[jb + 1]],
                                         xs[nsl], sem_s[nsl]),
                        pltpu.async_copy(x_ref.at[didx.at[jb + 1]],
                                         xd[nsl], sem_d[nsl]),
                    )
                pend[jb][0].wait()
                pend[jb][1].wait()
                if jb >= 2:
                    scat[jb - 2].wait()
                _xs = xs[sl]
                _xd = xd[sl]
                m = mm[sl]

                def _group(g, _, _jb=jb, _xs=_xs, _xd=_xd, m=m):
                    off = g * _L
                    vs = sidx[_jb, pl.ds(off, _L)]
                    vd = didx[_jb, pl.ds(off, _L)]
                    srows = []
                    for i in range(_L):
                        e = off + i
                        s0 = _xs[e, pl.ds(0, _L)]
                        s1 = _xs[e, pl.ds(_L, _L)]
                        srows.append((s0, s1))
                        p16 = (s0 * _xd[e, pl.ds(0, _L)] +
                               s1 * _xd[e, pl.ds(_L, _L)])
                        pbuf[pl.ds(i * _L, _L)] = plsc.cumsum(p16)
                    vdot = plsc.load_gather(pbuf, [lane15])
                    ex = jnp.where(vs != vd, jnp.exp(vdot), 0.0)
                    exb[_jb, pl.ds(off, _L)] = ex
                    for i in range(_L):
                        e = off + i
                        exi = ex[i]
                        s0, s1 = srows[i]
                        m[e, pl.ds(0, _L)] = exi * s0
                        m[e, pl.ds(_L, _L)] = exi * s1
                    return 0

                lax.fori_loop(0, _B // _L, _group, 0)
                scat[jb] = pltpu.async_copy(m, acc_n.at[didx.at[jb]],
                                            sem_m[sl], add=True)
                scat[(jb, "e")] = pltpu.async_copy(
                    exb.at[jb], accd.at[didx.at[jb]], sem_e, add=True)
            scat[rows_per_chunk - 2].wait()
            scat[rows_per_chunk - 1].wait()
            for jb in range(rows_per_chunk):
                scat[(jb, "e")].wait()
            if need_ex:
                pltpu.sync_copy(
                    exb, ex_ref.at[pl.ds(_p * epb + base_row, rows_per_chunk)])
            return 0

        lax.fori_loop(0, ept // _IC, _chunk, 0)

    plsc.subcore_barrier()

    dstripe = _NP // _NS
    pltpu.sync_copy(accd.at[pl.ds(s * dstripe, dstripe)],
                    denom_ref.at[c, pl.ds(s * dstripe, dstripe)])

    rows_per_tile = nout // _NS
    chunk = 112 if rows_per_tile % 125 else 125
    q0 = s * rows_per_tile
    def _cp(i, _):
        pltpu.sync_copy(acc_n.at[pl.ds(q0 + i * chunk, chunk)],
                        m0.at[pl.ds(0, chunk)])
        pltpu.sync_copy(m0.at[pl.ds(0, chunk)],
                        numer_ref.at[c, pl.ds(q0 + i * chunk, chunk)])
        return 0
    lax.fori_loop(0, rows_per_tile // chunk, _cp, 0)


def _make_conv(ept, nout, ndir, need_ex, epb):
    mesh = plsc.VectorSubcoreMesh(core_axis_name="c", subcore_axis_name="s",
                                  num_cores=_NC, num_subcores=_NS)
    outs = [
        jax.ShapeDtypeStruct((_NC, nout, _D), jnp.float32),
        jax.ShapeDtypeStruct((_NC, _NP), jnp.float32),
    ]
    if need_ex:
        outs.append(jax.ShapeDtypeStruct((ndir * epb, _B), jnp.float32))
    scratch = [
        pltpu.VMEM_SHARED((_N, _D), jnp.float32),
        pltpu.VMEM_SHARED((_NP,), jnp.float32),
        pltpu.VMEM((_IC // _B, _B), jnp.int32),
        pltpu.VMEM((_IC // _B, _B), jnp.int32),
        pltpu.VMEM((_B, _D), jnp.float32),
        pltpu.VMEM((_B, _D), jnp.float32),
        pltpu.VMEM((_B, _D), jnp.float32),
        pltpu.VMEM((_B, _D), jnp.float32),
        pltpu.VMEM((_B, _D), jnp.float32),
        pltpu.VMEM((_B, _D), jnp.float32),
        pltpu.VMEM((_IC // _B, _B), jnp.float32),
        pltpu.VMEM((_B * 2,), jnp.float32),
        pltpu.SemaphoreType.DMA,
        pltpu.SemaphoreType.DMA,
        pltpu.SemaphoreType.DMA,
        pltpu.SemaphoreType.DMA,
        pltpu.SemaphoreType.DMA,
        pltpu.SemaphoreType.DMA,
        pltpu.SemaphoreType.DMA,
        pltpu.SemaphoreType.DMA,
        pltpu.SemaphoreType.DMA,
    ]
    body = functools.partial(_conv_body, ept, nout, ndir, need_ex, epb)
    return pl.kernel(body, out_type=tuple(outs), mesh=mesh,
                     compiler_params=pltpu.CompilerParams(**_SC_PARAMS),
                     scratch_types=scratch)


def _alpha_body(ept, epb, den_ref, src_ref, dst_ref, ex_ref, al_ref,
                denl, didx0, didx1, exb0, exb1, alb0, alb1,
                sem_i0, sem_i1, sem_e0, sem_e1, sem_o0, sem_o1):
    c = lax.axis_index("c")
    s = lax.axis_index("s")
    tid = c * _NS + s
    pltpu.sync_copy(den_ref, denl)
    rows_per_chunk = _IC // _B
    didx = (didx0, didx1)
    exb = (exb0, exb1)
    alb = (alb0, alb1)
    sem_i = (sem_i0, sem_i1)
    sem_e = (sem_e0, sem_e1)
    sem_o = (sem_o0, sem_o1)
    nchunk = ept // _IC

    def _do(base, _p, sl):
        for jb in range(rows_per_chunk):
            def _grp(g, _, _jb=jb, _sl=sl):
                off = g * _L
                vd = didx[_sl][_jb, pl.ds(off, _L)]
                dv = plsc.load_gather(denl, [vd])
                ev = exb[_sl][_jb, pl.ds(off, _L)]
                alb[_sl][_jb, pl.ds(off, _L)] = ev / (dv + _EPS)
                return 0
            lax.fori_loop(0, _B // _L, _grp, 0)
        return pltpu.async_copy(
            alb[sl], al_ref.at[pl.ds(_p * epb + base, rows_per_chunk)],
            sem_o[sl])

    for p in range(2):
        dref = dst_ref if p == 0 else src_ref

        def _pair(q, _, _p=p, _dref=dref):
            baseA = tid * (ept // _B) + (2 * q) * rows_per_chunk
            baseB = baseA + rows_per_chunk
            ldA = (pltpu.async_copy(_dref.at[pl.ds(baseA, rows_per_chunk)],
                                    didx[0], sem_i[0]),
                   pltpu.async_copy(
                       ex_ref.at[pl.ds(_p * epb + baseA, rows_per_chunk)],
                       exb[0], sem_e[0]))
            ldB = (pltpu.async_copy(_dref.at[pl.ds(baseB, rows_per_chunk)],
                                    didx[1], sem_i[1]),
                   pltpu.async_copy(
                       ex_ref.at[pl.ds(_p * epb + baseB, rows_per_chunk)],
                       exb[1], sem_e[1]))
            ldA[0].wait(); ldA[1].wait()
            stA = _do(baseA, _p, 0)
            ldB[0].wait(); ldB[1].wait()
            stB = _do(baseB, _p, 1)
            stA.wait(); stB.wait()
            return 0

        lax.fori_loop(0, nchunk // 2, _pair, 0)
        if nchunk % 2:
            base = tid * (ept // _B) + (nchunk - 1) * rows_per_chunk
            pltpu.async_copy(dref.at[pl.ds(base, rows_per_chunk)],
                             didx[0], sem_i[0]).wait()
            pltpu.async_copy(ex_ref.at[pl.ds(p * epb + base, rows_per_chunk)],
                             exb[0], sem_e[0]).wait()
            _do(base, p, 0).wait()


def _make_alpha(ept, epb):
    mesh = plsc.VectorSubcoreMesh(core_axis_name="c", subcore_axis_name="s",
                                  num_cores=_NC, num_subcores=_NS)
    scratch = [
        pltpu.VMEM((_N,), jnp.float32),
        pltpu.VMEM((_IC // _B, _B), jnp.int32),
        pltpu.VMEM((_IC // _B, _B), jnp.int32),
        pltpu.VMEM((_IC // _B, _B), jnp.float32),
        pltpu.VMEM((_IC // _B, _B), jnp.float32),
        pltpu.VMEM((_IC // _B, _B), jnp.float32),
        pltpu.VMEM((_IC // _B, _B), jnp.float32),
    ] + [pltpu.SemaphoreType.DMA] * 6
    body = functools.partial(_alpha_body, ept, epb)
    return pl.kernel(body,
                     out_type=jax.ShapeDtypeStruct((2 * epb, _B), jnp.float32),
                     mesh=mesh,
                     compiler_params=pltpu.CompilerParams(**_SC_PARAMS),
                     scratch_types=scratch)


_RB = 1000


def _prep_feats_body(f_ref, w_ref, b_ref, o_ref):
    y = lax.dot_general(f_ref[...], w_ref[...],
                        (((1,), (1,)), ((), ())),
                        preferred_element_type=jnp.float32)
    y = y + b_ref[...]
    y = jnp.where(y > 0, y, 0.01 * y)
    n = jnp.sqrt(jnp.sum(y * y, axis=1, keepdims=True))
    o_ref[...] = y / jnp.maximum(n, 1e-12)


def _norm_body(p_ref, o_ref):
    p = p_ref[...]
    n = jnp.sqrt(jnp.sum(p * p, axis=1, keepdims=True))
    o_ref[...] = p / jnp.maximum(n, 1e-12)


def _route_body(p_ref, n_ref, d_ref, o_ref):
    num = n_ref[0] + n_ref[1]
    den = d_ref[0, 0, 0, :] + d_ref[1, 0, 0, :]
    out = num / (den[:, None] + _EPS)
    p2 = p_ref[...] + out
    nn = jnp.sqrt(jnp.sum(p2 * p2, axis=1, keepdims=True))
    o_ref[...] = p2 / jnp.maximum(nn, 1e-12)


def _final_body(x_ref, n_ref, d_ref, y_ref, dt_ref):
    num = n_ref[0] + n_ref[1]
    den = d_ref[0, 0, 0, :] + d_ref[1, 0, 0, :]
    out = num / (den[:, None] + _EPS)
    out = jnp.where(out > 0, out, 0.01 * out)
    y_ref[...] = x_ref[...] + out
    dt_ref[...] = jnp.broadcast_to(den[None, None, None, :],
                                   (8, 1, 1, den.shape[0]))


def kernel(feature, edge_index, preference, W, b):
    nu = preference.shape[0]
    ni = feature.shape[0]
    n_nodes = nu + ni
    e = edge_index.shape[1]

    ep = -(-e // (_NW * _IC)) * (_NW * _IC)
    epb = ep // _B
    ept = ep // _NW
    pad = ep - e
    src = jnp.concatenate([edge_index[0], jnp.zeros((pad,), jnp.int32)])
    dst = jnp.concatenate([edge_index[1], jnp.zeros((pad,), jnp.int32)])
    src2d = src.reshape(epb, _B)
    dst2d = dst.reshape(epb, _B)

    feats = pl.pallas_call(
        _prep_feats_body,
        grid=(ni // _RB,),
        in_specs=[
            pl.BlockSpec((_RB, feature.shape[1]), lambda i: (i, 0)),
            pl.BlockSpec(W.shape, lambda i: (0, 0)),
            pl.BlockSpec((1, _D), lambda i: (0, 0)),
        ],
        out_specs=pl.BlockSpec((_RB, _D), lambda i: (i, 0)),
        out_shape=jax.ShapeDtypeStruct((ni, _D), jnp.float32),
    )(feature, W, b.reshape(1, _D))

    pref = pl.pallas_call(
        _norm_body,
        grid=(nu // _RB,),
        in_specs=[pl.BlockSpec((_RB, _D), lambda i: (i, 0))],
        out_specs=pl.BlockSpec((_RB, _D), lambda i: (i, 0)),
        out_shape=jax.ShapeDtypeStruct((nu, _D), jnp.float32),
    )(preference)

    conv_route = _make_conv(ept, _NOUT_U, 1, False, epb)
    route = pl.pallas_call(
        _route_body,
        grid=(nu // _RB,),
        in_specs=[
            pl.BlockSpec((_RB, _D), lambda i: (i, 0)),
            pl.BlockSpec((_NC, _RB, _D), lambda i: (0, i, 0)),
            pl.BlockSpec((_NC, 1, 1, _RB), lambda i: (0, i, 0, 0)),
        ],
        out_specs=pl.BlockSpec((_RB, _D), lambda i: (i, 0)),
        out_shape=jax.ShapeDtypeStruct((nu, _D), jnp.float32),
    )

    for _ in range(2):
        x = jnp.concatenate([pref, feats], axis=0)
        numer_p, denom_p = conv_route(x, src2d, dst2d)
        d3 = denom_p[:, :nu].reshape(_NC, nu // _RB, 1, _RB)
        pref = route(pref, numer_p[:, :nu], d3)

    x = jnp.concatenate([pref, feats], axis=0)
    conv_final = _make_conv(ept, n_nodes, 2, True, epb)
    numer_p, denom_p, ex2d = conv_final(x, src2d, dst2d)

    y, dt = pl.pallas_call(
        _final_body,
        grid=(n_nodes // _RB,),
        in_specs=[
            pl.BlockSpec((_RB, _D), lambda i: (i, 0)),
            pl.BlockSpec((_NC, _RB, _D), lambda i: (0, i, 0)),
            pl.BlockSpec((_NC, 1, 1, _RB), lambda i: (0, i, 0, 0)),
        ],
        out_specs=[
            pl.BlockSpec((_RB, _D), lambda i: (i, 0)),
            pl.BlockSpec((8, 1, 1, _RB), lambda i: (0, i, 0, 0)),
        ],
        out_shape=[
            jax.ShapeDtypeStruct((n_nodes, _D), jnp.float32),
            jax.ShapeDtypeStruct((8, n_nodes // _RB, 1, _RB), jnp.float32),
        ],
    )(x, numer_p, denom_p[:, :n_nodes].reshape(_NC, n_nodes // _RB, 1, _RB))

    al2d = _make_alpha(ept, epb)(dt[0].reshape(n_nodes), src2d, dst2d, ex2d)
    al = al2d.reshape(-1)
    alpha = jnp.concatenate([al[:e], al[ep:ep + e]])[:, None]
    return (y, alpha)
